# Initial kernel scaffold; baseline (speedup 1.0000x reference)
import jax
import jax.numpy as jnp
from jax.experimental import pallas as pl


def kernel(x, edge_index, edge_attr, w1, root1, bias1, w2, root2, bias2):
    raise NotImplementedError("write your pallas kernel here")


# speedup vs baseline: 4.7664x; 4.7664x over previous
import functools

import jax
import jax.numpy as jnp
from jax import lax
from jax.experimental import pallas as pl
from jax.experimental.pallas import tpu as pltpu
from jax.experimental.pallas import tpu_sc as plsc

_HIGH = lax.Precision.HIGHEST

N_NODES = 10000
N_EDGES = 320000
NC = 2
NS = 16
NW = NC * NS
EPW = N_EDGES // NW
CHUNK = 80
NCHUNK = EPW // CHUNK
N_PAD = 10240
RPT = N_PAD // NS
ZROWS = 128
ACC_W = 32


def _sc_scatter_body(h_half, with_count,
                     u_hbm, src_hbm, dst_hbm, t_hbm, out_hbm,
                     src_v, dst_v, t_v, rows_v, msg_v, zb_v, acc_sh, sem):
    cid = lax.axis_index("c")
    sid = lax.axis_index("s")
    wid = sid * NC + cid
    zvec = jnp.zeros((16,), jnp.float32)

    def _zero_zb(r, c):
        zb_v[r, pl.ds(0, 16)] = zvec
        zb_v[r, pl.ds(16, 16)] = zvec
        return c
    lax.fori_loop(0, ZROWS, _zero_zb, 0)
    for r in range(RPT // ZROWS):
        pltpu.sync_copy(zb_v, acc_sh.at[pl.ds(sid * RPT + r * ZROWS, ZROWS)])

    if with_count:
        cnt_vec = (lax.iota(jnp.int32, 16) == 0).astype(jnp.float32)
        for e in range(CHUNK):
            msg_v[e, pl.ds(h_half, 16)] = cnt_vec
    plsc.subcore_barrier()

    def _chunk(g, c):
        base = wid * EPW + g * CHUNK
        pltpu.sync_copy(src_hbm.at[pl.ds(base, CHUNK)], src_v)
        pltpu.sync_copy(dst_hbm.at[pl.ds(base, CHUNK)], dst_v)
        pltpu.sync_copy(t_hbm.at[pl.ds(base, CHUNK)], t_v)
        pltpu.async_copy(u_hbm.at[src_v], rows_v, sem).wait()
        for jg in range(CHUNK // 16):
            tv = t_v[pl.ds(jg * 16, 16)]
            for j in range(16):
                e = jg * 16 + j
                for fb in range(h_half // 16):
                    z0 = rows_v[e, pl.ds(fb * 16, 16)]
                    z1 = rows_v[e, pl.ds(h_half + fb * 16, 16)]
                    msg_v[e, pl.ds(fb * 16, 16)] = z0 + tv[j] * z1
        pltpu.sync_copy(msg_v, acc_sh.at[dst_v], add=True)
        return c
    lax.fori_loop(0, NCHUNK, _chunk, 0)

    plsc.subcore_barrier()
    pltpu.sync_copy(acc_sh.at[pl.ds(sid * RPT, RPT)],
                    out_hbm.at[cid, pl.ds(sid * RPT, RPT)])


def _make_sc_scatter(h_half, with_count):
    mesh = plsc.VectorSubcoreMesh(core_axis_name="c", subcore_axis_name="s")
    return pl.kernel(
        functools.partial(_sc_scatter_body, h_half, with_count),
        out_type=jax.ShapeDtypeStruct((NC, N_PAD, ACC_W), jnp.float32),
        mesh=mesh,
        scratch_types=[
            pltpu.VMEM((CHUNK,), jnp.int32),
            pltpu.VMEM((CHUNK,), jnp.int32),
            pltpu.VMEM((CHUNK,), jnp.float32),
            pltpu.VMEM((CHUNK, 2 * h_half), jnp.float32),
            pltpu.VMEM((CHUNK, ACC_W), jnp.float32),
            pltpu.VMEM((ZROWS, ACC_W), jnp.float32),
            pltpu.VMEM_SHARED((N_PAD, ACC_W), jnp.float32),
            pltpu.SemaphoreType.DMA,
        ],
        compiler_params=pltpu.CompilerParams(
            use_tc_tiling_on_sc=False, needs_layout_passes=False),
    )


BR = 1000


def _tc_pre_body(x_ref, w1_ref, u_ref):
    w_cat = jnp.concatenate([w1_ref[0], w1_ref[1] - w1_ref[0]], axis=1)
    u_ref[...] = jnp.dot(x_ref[...], w_cat, precision=_HIGH)


def _tc_mid_body(p_ref, x_ref, root1_ref, bias1_ref, w2_ref, root2_ref,
                 bias2_ref, u2_ref, r2_ref, cnt_ref):
    psum = p_ref[0] + p_ref[1]
    cnt = jnp.maximum(psum[:, 16:17], 1.0)
    pre = (psum[:, :16] / cnt
           + jnp.dot(x_ref[...], root1_ref[...], precision=_HIGH)
           + bias1_ref[...])
    h = jnp.where(pre > 0, pre, jnp.exp(jnp.minimum(pre, 0.0)) - 1.0)
    w_cat = jnp.concatenate([w2_ref[0], w2_ref[1] - w2_ref[0]], axis=1)
    u2_ref[...] = jnp.dot(h, w_cat, precision=_HIGH)
    r2_ref[...] = jnp.dot(h, root2_ref[...], precision=_HIGH) + bias2_ref[...]
    cnt_ref[...] = cnt


def _tc_post_body(q_ref, cnt_ref, r2_ref, o_ref):
    logits = (q_ref[0] + q_ref[1]) / cnt_ref[...] + r2_ref[...]
    m = jnp.max(logits, axis=1, keepdims=True)
    lse = jnp.log(jnp.sum(jnp.exp(logits - m), axis=1, keepdims=True)) + m
    o_ref[...] = logits - lse


def kernel(x, edge_index, edge_attr, w1, root1, bias1, w2, root2, bias2):
    n, f_in = x.shape
    hid = w1.shape[2]
    ncls = w2.shape[2]
    src = edge_index[0]
    dst = edge_index[1]
    t = edge_attr[:, 0]

    grid = (n // BR,)
    u1 = pl.pallas_call(
        _tc_pre_body,
        grid=grid,
        in_specs=[
            pl.BlockSpec((BR, f_in), lambda i: (i, 0)),
            pl.BlockSpec((2, f_in, hid), lambda i: (0, 0, 0)),
        ],
        out_specs=pl.BlockSpec((BR, 2 * hid), lambda i: (i, 0)),
        out_shape=jax.ShapeDtypeStruct((n, 2 * hid), jnp.float32),
    )(x, w1)

    p1 = _make_sc_scatter(hid, True)(u1, src, dst, t)

    u2, r2, cnt = pl.pallas_call(
        _tc_mid_body,
        grid=grid,
        in_specs=[
            pl.BlockSpec((2, BR, ACC_W), lambda i: (0, i, 0)),
            pl.BlockSpec((BR, f_in), lambda i: (i, 0)),
            pl.BlockSpec((f_in, hid), lambda i: (0, 0)),
            pl.BlockSpec((hid,), lambda i: (0,)),
            pl.BlockSpec((2, hid, ncls), lambda i: (0, 0, 0)),
            pl.BlockSpec((hid, ncls), lambda i: (0, 0)),
            pl.BlockSpec((ncls,), lambda i: (0,)),
        ],
        out_specs=(
            pl.BlockSpec((BR, 2 * ncls), lambda i: (i, 0)),
            pl.BlockSpec((BR, ncls), lambda i: (i, 0)),
            pl.BlockSpec((BR, 1), lambda i: (i, 0)),
        ),
        out_shape=(
            jax.ShapeDtypeStruct((n, 2 * ncls), jnp.float32),
            jax.ShapeDtypeStruct((n, ncls), jnp.float32),
            jax.ShapeDtypeStruct((n, 1), jnp.float32),
        ),
    )(p1, x, root1, bias1, w2, root2, bias2)

    p2 = _make_sc_scatter(ncls, False)(u2, src, dst, t)

    out = pl.pallas_call(
        _tc_post_body,
        grid=grid,
        in_specs=[
            pl.BlockSpec((2, BR, ACC_W), lambda i: (0, i, 0)),
            pl.BlockSpec((BR, 1), lambda i: (i, 0)),
            pl.BlockSpec((BR, ncls), lambda i: (i, 0)),
        ],
        out_specs=pl.BlockSpec((BR, ncls), lambda i: (i, 0)),
        out_shape=jax.ShapeDtypeStruct((n, ncls), jnp.float32),
    )(p2, cnt, r2)
    return out


# speedup vs baseline: 5.9721x; 1.2530x over previous
import functools

import jax
import jax.numpy as jnp
from jax import lax
from jax.experimental import pallas as pl
from jax.experimental.pallas import tpu as pltpu
from jax.experimental.pallas import tpu_sc as plsc

_HIGH = lax.Precision.HIGHEST

N_NODES = 10000
N_EDGES = 320000
NC = 2
NS = 16
NW = NC * NS
CHUNK = 128
NCHUNK = 80
EPW = NCHUNK * CHUNK
E_PAD = NW * EPW
N_PAD = 10240
RPT = N_PAD // NS
ZROWS = 128
ACC_W = 32


def _sc_scatter_body(h_half, with_count,
                     u_hbm, src_hbm, dst_hbm, t_hbm, out_hbm,
                     src_v, dst_v, t_v, rows_v, msg_v, zb_v, acc_sh,
                     gsem0, gsem1, ssem0, ssem1):
    cid = lax.axis_index("c")
    sid = lax.axis_index("s")
    wid = sid * NC + cid
    zvec = jnp.zeros((16,), jnp.float32)
    gsem = (gsem0, gsem1)
    ssem = (ssem0, ssem1)

    def _zero_zb(r, c):
        zb_v[r, pl.ds(0, 16)] = zvec
        zb_v[r, pl.ds(16, 16)] = zvec
        return c
    lax.fori_loop(0, ZROWS, _zero_zb, 0)
    for r in range(RPT // ZROWS):
        pltpu.sync_copy(zb_v, acc_sh.at[pl.ds(sid * RPT + r * ZROWS, ZROWS)])

    pltpu.sync_copy(src_hbm.at[wid], src_v)
    pltpu.sync_copy(dst_hbm.at[wid], dst_v)
    pltpu.sync_copy(t_hbm.at[wid], t_v)

    if with_count:
        cnt_vec = (lax.iota(jnp.int32, 16) == 0).astype(jnp.float32)
        for b in range(2):
            for e in range(CHUNK):
                msg_v[b, e, pl.ds(h_half, 16)] = cnt_vec
    plsc.subcore_barrier()

    pltpu.async_copy(u_hbm.at[src_v.at[0]], rows_v.at[0], gsem[0]).wait()

    def _pair(gg, c):
        for b in range(2):
            g = 2 * gg + b
            nxt = 1 - b

            @pl.when(g + 1 < NCHUNK)
            def _issue():
                pltpu.async_copy(u_hbm.at[src_v.at[g + 1]], rows_v.at[nxt],
                                 gsem[nxt])

            @pl.when(g >= 2)
            def _drain_prev():
                pltpu.make_async_copy(msg_v.at[b], acc_sh.at[dst_v.at[g]],
                                      ssem[b]).wait()

            for jg in range(CHUNK // 16):
                tv = t_v[g, pl.ds(jg * 16, 16)]
                for j in range(16):
                    e = jg * 16 + j
                    for fb in range(h_half // 16):
                        z0 = rows_v[b, e, pl.ds(fb * 16, 16)]
                        z1 = rows_v[b, e, pl.ds(h_half + fb * 16, 16)]
                        msg_v[b, e, pl.ds(fb * 16, 16)] = z0 + tv[j] * z1
            pltpu.async_copy(msg_v.at[b], acc_sh.at[dst_v.at[g]], ssem[b],
                             add=True)

            @pl.when(g + 1 < NCHUNK)
            def _wait_gather():
                pltpu.make_async_copy(u_hbm.at[src_v.at[g + 1]],
                                      rows_v.at[nxt], gsem[nxt]).wait()
        return c
    lax.fori_loop(0, NCHUNK // 2, _pair, 0)

    for b in range(2):
        pltpu.make_async_copy(msg_v.at[b], acc_sh.at[dst_v.at[0]],
                              ssem[b]).wait()

    plsc.subcore_barrier()
    pltpu.sync_copy(acc_sh.at[pl.ds(sid * RPT, RPT)],
                    out_hbm.at[cid, pl.ds(sid * RPT, RPT)])


def _make_sc_scatter(h_half, with_count):
    mesh = plsc.VectorSubcoreMesh(core_axis_name="c", subcore_axis_name="s")
    return pl.kernel(
        functools.partial(_sc_scatter_body, h_half, with_count),
        out_type=jax.ShapeDtypeStruct((NC, N_PAD, ACC_W), jnp.float32),
        mesh=mesh,
        scratch_types=[
            pltpu.VMEM((NCHUNK, CHUNK), jnp.int32),
            pltpu.VMEM((NCHUNK, CHUNK), jnp.int32),
            pltpu.VMEM((NCHUNK, CHUNK), jnp.float32),
            pltpu.VMEM((2, CHUNK, 2 * h_half), jnp.float32),
            pltpu.VMEM((2, CHUNK, ACC_W), jnp.float32),
            pltpu.VMEM((ZROWS, ACC_W), jnp.float32),
            pltpu.VMEM_SHARED((N_PAD, ACC_W), jnp.float32),
            pltpu.SemaphoreType.DMA,
            pltpu.SemaphoreType.DMA,
            pltpu.SemaphoreType.DMA,
            pltpu.SemaphoreType.DMA,
        ],
        compiler_params=pltpu.CompilerParams(
            use_tc_tiling_on_sc=False, needs_layout_passes=False),
    )


BR = 1000


def _tc_pre_body(x_ref, w1_ref, u_ref):
    w_cat = jnp.concatenate([w1_ref[0], w1_ref[1] - w1_ref[0]], axis=1)
    u_ref[...] = jnp.dot(x_ref[...], w_cat, precision=_HIGH)


def _tc_mid_body(p_ref, x_ref, root1_ref, bias1_ref, w2_ref, root2_ref,
                 bias2_ref, u2_ref, r2_ref, cnt_ref):
    psum = p_ref[0] + p_ref[1]
    cnt = jnp.maximum(psum[:, 16:17], 1.0)
    pre = (psum[:, :16] / cnt
           + jnp.dot(x_ref[...], root1_ref[...], precision=_HIGH)
           + bias1_ref[...])
    h = jnp.where(pre > 0, pre, jnp.exp(jnp.minimum(pre, 0.0)) - 1.0)
    w_cat = jnp.concatenate([w2_ref[0], w2_ref[1] - w2_ref[0]], axis=1)
    u2_ref[...] = jnp.dot(h, w_cat, precision=_HIGH)
    r2_ref[...] = jnp.dot(h, root2_ref[...], precision=_HIGH) + bias2_ref[...]
    cnt_ref[...] = cnt


def _tc_post_body(q_ref, cnt_ref, r2_ref, o_ref):
    logits = (q_ref[0] + q_ref[1]) / cnt_ref[...] + r2_ref[...]
    m = jnp.max(logits, axis=1, keepdims=True)
    lse = jnp.log(jnp.sum(jnp.exp(logits - m), axis=1, keepdims=True)) + m
    o_ref[...] = logits - lse


def _pad3(a, fill):
    e = a.shape[0]
    a = jnp.concatenate([a, jnp.full((E_PAD - e,), fill, a.dtype)])
    return a.reshape(NW, NCHUNK, CHUNK)


def kernel(x, edge_index, edge_attr, w1, root1, bias1, w2, root2, bias2):
    n, f_in = x.shape
    hid = w1.shape[2]
    ncls = w2.shape[2]
    src = _pad3(edge_index[0], 0)
    dst = _pad3(edge_index[1], N_PAD - 1)
    t = _pad3(edge_attr[:, 0], 0)

    grid = (n // BR,)
    u1 = pl.pallas_call(
        _tc_pre_body,
        grid=grid,
        in_specs=[
            pl.BlockSpec((BR, f_in), lambda i: (i, 0)),
            pl.BlockSpec((2, f_in, hid), lambda i: (0, 0, 0)),
        ],
        out_specs=pl.BlockSpec((BR, 2 * hid), lambda i: (i, 0)),
        out_shape=jax.ShapeDtypeStruct((n, 2 * hid), jnp.float32),
    )(x, w1)

    p1 = _make_sc_scatter(hid, True)(u1, src, dst, t)

    u2, r2, cnt = pl.pallas_call(
        _tc_mid_body,
        grid=grid,
        in_specs=[
            pl.BlockSpec((2, BR, ACC_W), lambda i: (0, i, 0)),
            pl.BlockSpec((BR, f_in), lambda i: (i, 0)),
            pl.BlockSpec((f_in, hid), lambda i: (0, 0)),
            pl.BlockSpec((hid,), lambda i: (0,)),
            pl.BlockSpec((2, hid, ncls), lambda i: (0, 0, 0)),
            pl.BlockSpec((hid, ncls), lambda i: (0, 0)),
            pl.BlockSpec((ncls,), lambda i: (0,)),
        ],
        out_specs=(
            pl.BlockSpec((BR, 2 * ncls), lambda i: (i, 0)),
            pl.BlockSpec((BR, ncls), lambda i: (i, 0)),
            pl.BlockSpec((BR, 1), lambda i: (i, 0)),
        ),
        out_shape=(
            jax.ShapeDtypeStruct((n, 2 * ncls), jnp.float32),
            jax.ShapeDtypeStruct((n, ncls), jnp.float32),
            jax.ShapeDtypeStruct((n, 1), jnp.float32),
        ),
    )(p1, x, root1, bias1, w2, root2, bias2)

    p2 = _make_sc_scatter(ncls, False)(u2, src, dst, t)

    out = pl.pallas_call(
        _tc_post_body,
        grid=grid,
        in_specs=[
            pl.BlockSpec((2, BR, ACC_W), lambda i: (0, i, 0)),
            pl.BlockSpec((BR, 1), lambda i: (i, 0)),
            pl.BlockSpec((BR, ncls), lambda i: (i, 0)),
        ],
        out_specs=pl.BlockSpec((BR, ncls), lambda i: (i, 0)),
        out_shape=jax.ShapeDtypeStruct((n, ncls), jnp.float32),
    )(p2, cnt, r2)
    return out


# speedup vs baseline: 9.6214x; 1.6111x over previous
import functools

import jax
import jax.numpy as jnp
from jax import lax
from jax.experimental import pallas as pl
from jax.experimental.pallas import tpu as pltpu
from jax.experimental.pallas import tpu_sc as plsc

_HIGH = lax.Precision.HIGHEST

N_NODES = 10000
N_EDGES = 320000
NC = 2
NS = 16
NW = NC * NS
CHUNK = 128
CPP = 160
G_CHUNKS = NS * CPP
E_PAD = G_CHUNKS * CHUNK
N_PAD = 10240
RPT = N_PAD // NS
ZROWS = 128
ACC_W = 32

SPLIT_L1 = (120, 40)
SPLIT_L2 = (104, 56)


def _sc_scatter_body(mode, a, b,
                     u_hbm, src_hbm, dst_hbm, t_hbm, out_hbm,
                     src_v, dst_v, t_v, rows_v, msg_v, zb_v, acc_sh,
                     gsem0, gsem1, ssem0, ssem1):
    cid = lax.axis_index("c")
    sid = lax.axis_index("s")
    zvec = jnp.zeros((16,), jnp.float32)
    gsem = (gsem0, gsem1)
    ssem = (ssem0, ssem1)
    amax = max(a, b)
    start = cid * (NS * a) + sid * (a + cid * (b - a))
    npairs = a // 2 + cid * ((b - a) // 2)
    ncnt = 2 * npairs

    def _zero_zb(r, c):
        zb_v[r, pl.ds(0, 16)] = zvec
        zb_v[r, pl.ds(16, 16)] = zvec
        return c
    lax.fori_loop(0, ZROWS, _zero_zb, 0)
    for r in range(RPT // ZROWS):
        pltpu.sync_copy(zb_v, acc_sh.at[pl.ds(sid * RPT + r * ZROWS, ZROWS)])

    pltpu.sync_copy(src_hbm.at[pl.ds(start, amax)], src_v)
    pltpu.sync_copy(dst_hbm.at[pl.ds(start, amax)], dst_v)
    pltpu.sync_copy(t_hbm.at[pl.ds(start, amax)], t_v)

    if mode == 1:
        cnt_vec = (lax.iota(jnp.int32, 16) == 0).astype(jnp.float32)
        for s in range(2):
            for e in range(CHUNK):
                msg_v[s, e, pl.ds(16, 16)] = cnt_vec
    plsc.subcore_barrier()

    pltpu.async_copy(u_hbm.at[src_v.at[0]], rows_v.at[0], gsem[0]).wait()

    def _pair(gg, c):
        for s in range(2):
            g = 2 * gg + s
            nxt = 1 - s

            @pl.when(g + 1 < ncnt)
            def _issue():
                pltpu.async_copy(u_hbm.at[src_v.at[g + 1]], rows_v.at[nxt],
                                 gsem[nxt])

            @pl.when(g >= 2)
            def _drain_prev():
                pltpu.make_async_copy(msg_v.at[s], acc_sh.at[dst_v.at[g]],
                                      ssem[s]).wait()

            for jg in range(CHUNK // 16):
                tv = t_v[g, pl.ds(jg * 16, 16)]
                for j in range(16):
                    e = jg * 16 + j
                    if mode == 1:
                        z0 = rows_v[s, e, pl.ds(0, 16)]
                        z1 = rows_v[s, e, pl.ds(16, 16)]
                        msg_v[s, e, pl.ds(0, 16)] = z0 + tv[j] * z1
                    else:
                        hrow = rows_v[s, e, pl.ds(0, 16)]
                        msg_v[s, e, pl.ds(0, 16)] = hrow
                        msg_v[s, e, pl.ds(16, 16)] = tv[j] * hrow
            pltpu.async_copy(msg_v.at[s], acc_sh.at[dst_v.at[g]], ssem[s],
                             add=True)

            @pl.when(g + 1 < ncnt)
            def _wait_gather():
                pltpu.make_async_copy(u_hbm.at[src_v.at[g + 1]],
                                      rows_v.at[nxt], gsem[nxt]).wait()
        return c
    lax.fori_loop(0, npairs, _pair, 0)

    for s in range(2):
        pltpu.make_async_copy(msg_v.at[s], acc_sh.at[dst_v.at[0]],
                              ssem[s]).wait()

    plsc.subcore_barrier()
    pltpu.sync_copy(acc_sh.at[pl.ds(sid * RPT, RPT)],
                    out_hbm.at[cid, pl.ds(sid * RPT, RPT)])


def _make_sc_scatter(mode, split):
    a, b = split
    amax = max(a, b)
    row_w = 32 if mode == 1 else 16
    mesh = plsc.VectorSubcoreMesh(core_axis_name="c", subcore_axis_name="s")
    return pl.kernel(
        functools.partial(_sc_scatter_body, mode, a, b),
        out_type=jax.ShapeDtypeStruct((NC, N_PAD, ACC_W), jnp.float32),
        mesh=mesh,
        scratch_types=[
            pltpu.VMEM((amax, CHUNK), jnp.int32),
            pltpu.VMEM((amax, CHUNK), jnp.int32),
            pltpu.VMEM((amax, CHUNK), jnp.float32),
            pltpu.VMEM((2, CHUNK, row_w), jnp.float32),
            pltpu.VMEM((2, CHUNK, ACC_W), jnp.float32),
            pltpu.VMEM((ZROWS, ACC_W), jnp.float32),
            pltpu.VMEM_SHARED((N_PAD, ACC_W), jnp.float32),
            pltpu.SemaphoreType.DMA,
            pltpu.SemaphoreType.DMA,
            pltpu.SemaphoreType.DMA,
            pltpu.SemaphoreType.DMA,
        ],
        compiler_params=pltpu.CompilerParams(
            use_tc_tiling_on_sc=False, needs_layout_passes=False),
    )


BR = 1000


def _tc_pre_body(x_ref, w1_ref, u_ref):
    w_cat = jnp.concatenate([w1_ref[0], w1_ref[1] - w1_ref[0]], axis=1)
    u_ref[...] = jnp.dot(x_ref[...], w_cat, precision=_HIGH)


def _tc_mid_body(p_ref, x_ref, root1_ref, bias1_ref, root2_ref,
                 bias2_ref, h_ref, r2_ref, cnt_ref):
    psum = p_ref[0] + p_ref[1]
    cnt = jnp.maximum(psum[:, 16:17], 1.0)
    pre = (psum[:, :16] / cnt
           + jnp.dot(x_ref[...], root1_ref[...], precision=_HIGH)
           + bias1_ref[...])
    h = jnp.where(pre > 0, pre, jnp.exp(jnp.minimum(pre, 0.0)) - 1.0)
    h_ref[...] = h
    r2_ref[...] = jnp.dot(h, root2_ref[...], precision=_HIGH) + bias2_ref[...]
    cnt_ref[...] = cnt


def _tc_post_body(q_ref, w2_ref, cnt_ref, r2_ref, o_ref):
    qsum = q_ref[0] + q_ref[1]
    sums2 = (jnp.dot(qsum[:, :16], w2_ref[0], precision=_HIGH)
             + jnp.dot(qsum[:, 16:], w2_ref[1] - w2_ref[0], precision=_HIGH))
    logits = sums2 / cnt_ref[...] + r2_ref[...]
    m = jnp.max(logits, axis=1, keepdims=True)
    lse = jnp.log(jnp.sum(jnp.exp(logits - m), axis=1, keepdims=True)) + m
    o_ref[...] = logits - lse


def _pad_chunks(arr, fill, alloc_chunks):
    e = arr.shape[0]
    pad = alloc_chunks * CHUNK - e
    arr = jnp.concatenate([arr, jnp.full((pad,), fill, arr.dtype)])
    return arr.reshape(alloc_chunks, CHUNK)


def kernel(x, edge_index, edge_attr, w1, root1, bias1, w2, root2, bias2):
    n, f_in = x.shape
    hid = w1.shape[2]
    ncls = w2.shape[2]
    alloc = G_CHUNKS + max(
        max(s) - min(s) for s in (SPLIT_L1, SPLIT_L2))
    src = _pad_chunks(edge_index[0], 0, alloc)
    dst = _pad_chunks(edge_index[1], N_PAD - 1, alloc)
    t = _pad_chunks(edge_attr[:, 0], 0.0, alloc)

    grid = (n // BR,)
    u1 = pl.pallas_call(
        _tc_pre_body,
        grid=grid,
        in_specs=[
            pl.BlockSpec((BR, f_in), lambda i: (i, 0)),
            pl.BlockSpec((2, f_in, hid), lambda i: (0, 0, 0)),
        ],
        out_specs=pl.BlockSpec((BR, 2 * hid), lambda i: (i, 0)),
        out_shape=jax.ShapeDtypeStruct((n, 2 * hid), jnp.float32),
    )(x, w1)

    p1 = _make_sc_scatter(1, SPLIT_L1)(u1, src, dst, t)

    h, r2, cnt = pl.pallas_call(
        _tc_mid_body,
        grid=grid,
        in_specs=[
            pl.BlockSpec((2, BR, ACC_W), lambda i: (0, i, 0)),
            pl.BlockSpec((BR, f_in), lambda i: (i, 0)),
            pl.BlockSpec((f_in, hid), lambda i: (0, 0)),
            pl.BlockSpec((hid,), lambda i: (0,)),
            pl.BlockSpec((hid, ncls), lambda i: (0, 0)),
            pl.BlockSpec((ncls,), lambda i: (0,)),
        ],
        out_specs=(
            pl.BlockSpec((BR, hid), lambda i: (i, 0)),
            pl.BlockSpec((BR, ncls), lambda i: (i, 0)),
            pl.BlockSpec((BR, 1), lambda i: (i, 0)),
        ),
        out_shape=(
            jax.ShapeDtypeStruct((n, hid), jnp.float32),
            jax.ShapeDtypeStruct((n, ncls), jnp.float32),
            jax.ShapeDtypeStruct((n, 1), jnp.float32),
        ),
    )(p1, x, root1, bias1, root2, bias2)

    p2 = _make_sc_scatter(2, SPLIT_L2)(h, src, dst, t)

    out = pl.pallas_call(
        _tc_post_body,
        grid=grid,
        in_specs=[
            pl.BlockSpec((2, BR, ACC_W), lambda i: (0, i, 0)),
            pl.BlockSpec((2, hid, ncls), lambda i: (0, 0, 0)),
            pl.BlockSpec((BR, 1), lambda i: (i, 0)),
            pl.BlockSpec((BR, ncls), lambda i: (i, 0)),
        ],
        out_specs=pl.BlockSpec((BR, ncls), lambda i: (i, 0)),
        out_shape=jax.ShapeDtypeStruct((n, ncls), jnp.float32),
    )(p2, w2, cnt, r2)
    return out


# speedup vs baseline: 11.5044x; 1.1957x over previous
import functools

import jax
import jax.numpy as jnp
from jax import lax
from jax.experimental import pallas as pl
from jax.experimental.pallas import tpu as pltpu
from jax.experimental.pallas import tpu_sc as plsc

_HIGH = lax.Precision.HIGHEST

N_NODES = 10000
N_EDGES = 320000
NC = 2
NS = 16
CHUNK = 128
G_CHUNKS = N_EDGES // CHUNK
N_PAD = 10240
RPT = N_PAD // NS
ZROWS = 128
ACC_W = 32

SPLIT_L1 = (112, 44)
SPLIT_L2 = (90, 66)


def _sc_scatter_body(mode, a, b,
                     u_hbm, ei_hbm, ea_hbm, out_hbm,
                     src_v, dst_v, t_v, rows_v, msg_v, zb_v, acc_sh,
                     gsem0, gsem1, ssem0, ssem1):
    cid = lax.axis_index("c")
    sid = lax.axis_index("s")
    zvec = jnp.zeros((16,), jnp.float32)
    gsem = (gsem0, gsem1)
    ssem = (ssem0, ssem1)
    a2 = a + 2
    start0 = sid * a + 2 * jnp.minimum(sid, 2)
    start1 = NS * a + 4 + sid * b
    start = jnp.where(cid == 0, start0, start1)
    npairs = jnp.where(cid == 0, a // 2 + (sid < 2), b // 2)
    ncnt = 2 * npairs

    def _zero_zb(r, c):
        zb_v[r, pl.ds(0, 16)] = zvec
        zb_v[r, pl.ds(16, 16)] = zvec
        return c
    lax.fori_loop(0, ZROWS, _zero_zb, 0)
    for r in range(RPT // ZROWS):
        pltpu.sync_copy(zb_v, acc_sh.at[pl.ds(sid * RPT + r * ZROWS, ZROWS)])

    e0 = start * CHUNK

    @pl.when(cid == 0)
    def _stage_a():
        pltpu.sync_copy(ei_hbm.at[pl.ds(0, 1), pl.ds(e0, a2 * CHUNK)],
                        src_v.at[pl.ds(0, 1), pl.ds(0, a2 * CHUNK)])
        pltpu.sync_copy(ei_hbm.at[pl.ds(1, 1), pl.ds(e0, a2 * CHUNK)],
                        dst_v.at[pl.ds(0, 1), pl.ds(0, a2 * CHUNK)])
        pltpu.sync_copy(ea_hbm.at[pl.ds(e0, a2 * CHUNK)],
                        t_v.at[pl.ds(0, a2 * CHUNK)])

    @pl.when(cid == 1)
    def _stage_b():
        pltpu.sync_copy(ei_hbm.at[pl.ds(0, 1), pl.ds(e0, b * CHUNK)],
                        src_v.at[pl.ds(0, 1), pl.ds(0, b * CHUNK)])
        pltpu.sync_copy(ei_hbm.at[pl.ds(1, 1), pl.ds(e0, b * CHUNK)],
                        dst_v.at[pl.ds(0, 1), pl.ds(0, b * CHUNK)])
        pltpu.sync_copy(ea_hbm.at[pl.ds(e0, b * CHUNK)],
                        t_v.at[pl.ds(0, b * CHUNK)])

    if mode == 1:
        cnt_vec = (lax.iota(jnp.int32, 16) == 0).astype(jnp.float32)
        for s in range(2):
            for e in range(CHUNK):
                msg_v[s, e, pl.ds(16, 16)] = cnt_vec
    plsc.subcore_barrier()

    pltpu.async_copy(u_hbm.at[src_v.at[0, pl.ds(0, CHUNK)]], rows_v.at[0],
                     gsem[0]).wait()

    def _pair(gg, c):
        for s in range(2):
            g = 2 * gg + s
            nxt = 1 - s

            @pl.when(g + 1 < ncnt)
            def _issue():
                pltpu.async_copy(
                    u_hbm.at[src_v.at[0, pl.ds((g + 1) * CHUNK, CHUNK)]],
                    rows_v.at[nxt], gsem[nxt])

            @pl.when(g >= 2)
            def _drain_prev():
                pltpu.make_async_copy(
                    msg_v.at[s],
                    acc_sh.at[dst_v.at[0, pl.ds(g * CHUNK, CHUNK)]],
                    ssem[s]).wait()

            for jg in range(CHUNK // 16):
                tv = t_v[pl.ds(g * CHUNK + jg * 16, 16)]
                for j in range(16):
                    e = jg * 16 + j
                    if mode == 1:
                        z0 = rows_v[s, e, pl.ds(0, 16)]
                        z1 = rows_v[s, e, pl.ds(16, 16)]
                        msg_v[s, e, pl.ds(0, 16)] = z0 + tv[j] * z1
                    else:
                        hrow = rows_v[s, e, pl.ds(0, 16)]
                        msg_v[s, e, pl.ds(0, 16)] = hrow
                        msg_v[s, e, pl.ds(16, 16)] = tv[j] * hrow
            pltpu.async_copy(msg_v.at[s],
                             acc_sh.at[dst_v.at[0, pl.ds(g * CHUNK, CHUNK)]],
                             ssem[s], add=True)

            @pl.when(g + 1 < ncnt)
            def _wait_gather():
                pltpu.make_async_copy(
                    u_hbm.at[src_v.at[0, pl.ds((g + 1) * CHUNK, CHUNK)]],
                    rows_v.at[nxt], gsem[nxt]).wait()
        return c
    lax.fori_loop(0, npairs, _pair, 0)

    for s in range(2):
        pltpu.make_async_copy(msg_v.at[s],
                              acc_sh.at[dst_v.at[0, pl.ds(0, CHUNK)]],
                              ssem[s]).wait()

    plsc.subcore_barrier()
    pltpu.sync_copy(acc_sh.at[pl.ds(sid * RPT, RPT)],
                    out_hbm.at[cid, pl.ds(sid * RPT, RPT)])


def _make_sc_scatter(mode, split):
    a, b = split
    ecap = (a + 2) * CHUNK
    row_w = 32 if mode == 1 else 16
    mesh = plsc.VectorSubcoreMesh(core_axis_name="c", subcore_axis_name="s")
    return pl.kernel(
        functools.partial(_sc_scatter_body, mode, a, b),
        out_type=jax.ShapeDtypeStruct((NC, N_PAD, ACC_W), jnp.float32),
        mesh=mesh,
        scratch_types=[
            pltpu.VMEM((1, ecap), jnp.int32),
            pltpu.VMEM((1, ecap), jnp.int32),
            pltpu.VMEM((ecap,), jnp.float32),
            pltpu.VMEM((2, CHUNK, row_w), jnp.float32),
            pltpu.VMEM((2, CHUNK, ACC_W), jnp.float32),
            pltpu.VMEM((ZROWS, ACC_W), jnp.float32),
            pltpu.VMEM_SHARED((N_PAD, ACC_W), jnp.float32),
            pltpu.SemaphoreType.DMA,
            pltpu.SemaphoreType.DMA,
            pltpu.SemaphoreType.DMA,
            pltpu.SemaphoreType.DMA,
        ],
        compiler_params=pltpu.CompilerParams(
            use_tc_tiling_on_sc=False, needs_layout_passes=False),
    )


BR = 1000


def _tc_pre_body(x_ref, w1_ref, root1_ref, u_ref, xr_ref):
    w_cat = jnp.concatenate([w1_ref[0], w1_ref[1] - w1_ref[0]], axis=1)
    xb = x_ref[...]
    u_ref[...] = jnp.dot(xb, w_cat, precision=_HIGH)
    xr_ref[...] = jnp.dot(xb, root1_ref[...], precision=_HIGH)


def _tc_mid_body(p_ref, xr_ref, bias1_ref, root2_ref,
                 bias2_ref, h_ref, r2_ref, cnt_ref):
    psum = p_ref[0] + p_ref[1]
    cnt = jnp.maximum(psum[:, 16:17], 1.0)
    pre = psum[:, :16] / cnt + xr_ref[...] + bias1_ref[...]
    h = jnp.where(pre > 0, pre, jnp.exp(jnp.minimum(pre, 0.0)) - 1.0)
    h_ref[...] = h
    r2_ref[...] = jnp.dot(h, root2_ref[...], precision=_HIGH) + bias2_ref[...]
    cnt_ref[...] = cnt


def _tc_post_body(q_ref, w2_ref, cnt_ref, r2_ref, o_ref):
    qsum = q_ref[0] + q_ref[1]
    sums2 = (jnp.dot(qsum[:, :16], w2_ref[0], precision=_HIGH)
             + jnp.dot(qsum[:, 16:], w2_ref[1] - w2_ref[0], precision=_HIGH))
    logits = sums2 / cnt_ref[...] + r2_ref[...]
    m = jnp.max(logits, axis=1, keepdims=True)
    lse = jnp.log(jnp.sum(jnp.exp(logits - m), axis=1, keepdims=True)) + m
    o_ref[...] = logits - lse


def kernel(x, edge_index, edge_attr, w1, root1, bias1, w2, root2, bias2):
    n, f_in = x.shape
    hid = w1.shape[2]
    ncls = w2.shape[2]

    grid = (n // BR,)
    u1, xr1 = pl.pallas_call(
        _tc_pre_body,
        grid=grid,
        in_specs=[
            pl.BlockSpec((BR, f_in), lambda i: (i, 0)),
            pl.BlockSpec((2, f_in, hid), lambda i: (0, 0, 0)),
            pl.BlockSpec((f_in, hid), lambda i: (0, 0)),
        ],
        out_specs=(
            pl.BlockSpec((BR, 2 * hid), lambda i: (i, 0)),
            pl.BlockSpec((BR, hid), lambda i: (i, 0)),
        ),
        out_shape=(
            jax.ShapeDtypeStruct((n, 2 * hid), jnp.float32),
            jax.ShapeDtypeStruct((n, hid), jnp.float32),
        ),
    )(x, w1, root1)

    t1d = edge_attr.reshape(-1)
    p1 = _make_sc_scatter(1, SPLIT_L1)(u1, edge_index, t1d)

    h, r2, cnt = pl.pallas_call(
        _tc_mid_body,
        grid=grid,
        in_specs=[
            pl.BlockSpec((2, BR, ACC_W), lambda i: (0, i, 0)),
            pl.BlockSpec((BR, hid), lambda i: (i, 0)),
            pl.BlockSpec((hid,), lambda i: (0,)),
            pl.BlockSpec((hid, ncls), lambda i: (0, 0)),
            pl.BlockSpec((ncls,), lambda i: (0,)),
        ],
        out_specs=(
            pl.BlockSpec((BR, hid), lambda i: (i, 0)),
            pl.BlockSpec((BR, ncls), lambda i: (i, 0)),
            pl.BlockSpec((BR, 1), lambda i: (i, 0)),
        ),
        out_shape=(
            jax.ShapeDtypeStruct((n, hid), jnp.float32),
            jax.ShapeDtypeStruct((n, ncls), jnp.float32),
            jax.ShapeDtypeStruct((n, 1), jnp.float32),
        ),
    )(p1, xr1, bias1, root2, bias2)

    p2 = _make_sc_scatter(2, SPLIT_L2)(h, edge_index, t1d)

    out = pl.pallas_call(
        _tc_post_body,
        grid=grid,
        in_specs=[
            pl.BlockSpec((2, BR, ACC_W), lambda i: (0, i, 0)),
            pl.BlockSpec((2, hid, ncls), lambda i: (0, 0, 0)),
            pl.BlockSpec((BR, 1), lambda i: (i, 0)),
            pl.BlockSpec((BR, ncls), lambda i: (i, 0)),
        ],
        out_specs=pl.BlockSpec((BR, ncls), lambda i: (i, 0)),
        out_shape=jax.ShapeDtypeStruct((n, ncls), jnp.float32),
    )(p2, w2, cnt, r2)
    return out


# speedup vs baseline: 12.8471x; 1.1167x over previous
import functools

import jax
import jax.numpy as jnp
from jax import lax
from jax.experimental import pallas as pl
from jax.experimental.pallas import tpu as pltpu
from jax.experimental.pallas import tpu_sc as plsc

_HIGH = lax.Precision.HIGHEST

N_NODES = 10000
N_EDGES = 320000
NC = 2
NS = 16
CHUNK = 128
G_CHUNKS = N_EDGES // CHUNK
N_PAD = 10240
RPT = N_PAD // NS
ZROWS = 128
ACC_W = 32

SPLIT_L1 = (82, 74)
SPLIT_L2 = (80, 76)


def _sc_scatter_body(mode, a, b,
                     u_hbm, ei_hbm, ea_hbm, out_hbm,
                     src_v, dst_v, t_v, rows_v, msg_v, zb_v, acc_sh,
                     gsem0, gsem1, ssem0, ssem1):
    cid = lax.axis_index("c")
    sid = lax.axis_index("s")
    zvec = jnp.zeros((16,), jnp.float32)
    gsem = (gsem0, gsem1)
    ssem = (ssem0, ssem1)
    a2 = a + 2
    start0 = sid * a + 2 * jnp.minimum(sid, 2)
    start1 = NS * a + 4 + sid * b
    start = jnp.where(cid == 0, start0, start1)
    npairs = jnp.where(cid == 0, a // 2 + (sid < 2), b // 2)
    ncnt = 2 * npairs

    def _zero_zb(r, c):
        zb_v[r, pl.ds(0, 16)] = zvec
        zb_v[r, pl.ds(16, 16)] = zvec
        return c
    lax.fori_loop(0, ZROWS, _zero_zb, 0)
    for r in range(RPT // ZROWS):
        pltpu.sync_copy(zb_v, acc_sh.at[pl.ds(sid * RPT + r * ZROWS, ZROWS)])

    e0 = start * CHUNK

    @pl.when(cid == 0)
    def _stage_a():
        pltpu.sync_copy(ei_hbm.at[pl.ds(0, 1), pl.ds(e0, a2 * CHUNK)],
                        src_v.at[pl.ds(0, 1), pl.ds(0, a2 * CHUNK)])
        pltpu.sync_copy(ei_hbm.at[pl.ds(1, 1), pl.ds(e0, a2 * CHUNK)],
                        dst_v.at[pl.ds(0, 1), pl.ds(0, a2 * CHUNK)])
        pltpu.sync_copy(ea_hbm.at[pl.ds(e0, a2 * CHUNK)],
                        t_v.at[pl.ds(0, a2 * CHUNK)])

    @pl.when(cid == 1)
    def _stage_b():
        pltpu.sync_copy(ei_hbm.at[pl.ds(0, 1), pl.ds(e0, b * CHUNK)],
                        src_v.at[pl.ds(0, 1), pl.ds(0, b * CHUNK)])
        pltpu.sync_copy(ei_hbm.at[pl.ds(1, 1), pl.ds(e0, b * CHUNK)],
                        dst_v.at[pl.ds(0, 1), pl.ds(0, b * CHUNK)])
        pltpu.sync_copy(ea_hbm.at[pl.ds(e0, b * CHUNK)],
                        t_v.at[pl.ds(0, b * CHUNK)])

    if mode == 1:
        cnt_vec = (lax.iota(jnp.int32, 16) == 0).astype(jnp.float32)
        for s in range(2):
            for e in range(CHUNK):
                msg_v[s, e, pl.ds(16, 16)] = cnt_vec
    plsc.subcore_barrier()

    pltpu.async_copy(u_hbm.at[src_v.at[0, pl.ds(0, CHUNK)]], rows_v.at[0],
                     gsem[0]).wait()

    def _pair(gg, c):
        for s in range(2):
            g = 2 * gg + s
            nxt = 1 - s

            @pl.when(g + 1 < ncnt)
            def _issue():
                pltpu.async_copy(
                    u_hbm.at[src_v.at[0, pl.ds((g + 1) * CHUNK, CHUNK)]],
                    rows_v.at[nxt], gsem[nxt])

            @pl.when(g >= 2)
            def _drain_prev():
                pltpu.make_async_copy(
                    msg_v.at[s],
                    acc_sh.at[dst_v.at[0, pl.ds(g * CHUNK, CHUNK)]],
                    ssem[s]).wait()

            for jg in range(CHUNK // 16):
                tv = t_v[pl.ds(g * CHUNK + jg * 16, 16)]
                for j in range(16):
                    e = jg * 16 + j
                    if mode == 1:
                        z0 = rows_v[s, e, pl.ds(0, 16)]
                        z1 = rows_v[s, e, pl.ds(16, 16)]
                        msg_v[s, e, pl.ds(0, 16)] = z0 + tv[j] * z1
                    else:
                        hrow = rows_v[s, e, pl.ds(0, 16)]
                        msg_v[s, e, pl.ds(0, 16)] = hrow
                        msg_v[s, e, pl.ds(16, 16)] = tv[j] * hrow
            pltpu.async_copy(msg_v.at[s],
                             acc_sh.at[dst_v.at[0, pl.ds(g * CHUNK, CHUNK)]],
                             ssem[s], add=True)

            @pl.when(g + 1 < ncnt)
            def _wait_gather():
                pltpu.make_async_copy(
                    u_hbm.at[src_v.at[0, pl.ds((g + 1) * CHUNK, CHUNK)]],
                    rows_v.at[nxt], gsem[nxt]).wait()
        return c
    lax.fori_loop(0, npairs, _pair, 0)

    for s in range(2):
        pltpu.make_async_copy(msg_v.at[s],
                              acc_sh.at[dst_v.at[0, pl.ds(0, CHUNK)]],
                              ssem[s]).wait()

    plsc.subcore_barrier()
    pltpu.sync_copy(acc_sh.at[pl.ds(sid * RPT, RPT)],
                    out_hbm.at[cid, pl.ds(sid * RPT, RPT)])


def _make_sc_scatter(mode, split):
    a, b = split
    ecap = (a + 2) * CHUNK
    row_w = 32 if mode == 1 else 16
    mesh = plsc.VectorSubcoreMesh(core_axis_name="c", subcore_axis_name="s")
    return pl.kernel(
        functools.partial(_sc_scatter_body, mode, a, b),
        out_type=jax.ShapeDtypeStruct((NC, N_PAD, ACC_W), jnp.float32),
        mesh=mesh,
        scratch_types=[
            pltpu.VMEM((1, ecap), jnp.int32),
            pltpu.VMEM((1, ecap), jnp.int32),
            pltpu.VMEM((ecap,), jnp.float32),
            pltpu.VMEM((2, CHUNK, row_w), jnp.float32),
            pltpu.VMEM((2, CHUNK, ACC_W), jnp.float32),
            pltpu.VMEM((ZROWS, ACC_W), jnp.float32),
            pltpu.VMEM_SHARED((N_PAD, ACC_W), jnp.float32),
            pltpu.SemaphoreType.DMA,
            pltpu.SemaphoreType.DMA,
            pltpu.SemaphoreType.DMA,
            pltpu.SemaphoreType.DMA,
        ],
        compiler_params=pltpu.CompilerParams(
            use_tc_tiling_on_sc=False, needs_layout_passes=False),
    )


BR = 1000


def _tc_pre_body(x_ref, w1_ref, root1_ref, u_ref, xr_ref):
    w_cat = jnp.concatenate([w1_ref[0], w1_ref[1] - w1_ref[0]], axis=1)
    xb = x_ref[...]
    u_ref[...] = jnp.dot(xb, w_cat, precision=_HIGH)
    xr_ref[...] = jnp.dot(xb, root1_ref[...], precision=_HIGH)


def _tc_mid_body(p_ref, xr_ref, bias1_ref, root2_ref,
                 bias2_ref, h_ref, r2_ref, cnt_ref):
    psum = p_ref[0] + p_ref[1]
    cnt = jnp.maximum(psum[:, 16:17], 1.0)
    pre = psum[:, :16] / cnt + xr_ref[...] + bias1_ref[...]
    h = jnp.where(pre > 0, pre, jnp.exp(jnp.minimum(pre, 0.0)) - 1.0)
    h_ref[...] = h
    r2_ref[...] = jnp.dot(h, root2_ref[...], precision=_HIGH) + bias2_ref[...]
    cnt_ref[...] = cnt


def _tc_post_body(q_ref, w2_ref, cnt_ref, r2_ref, o_ref):
    qsum = q_ref[0] + q_ref[1]
    sums2 = (jnp.dot(qsum[:, :16], w2_ref[0], precision=_HIGH)
             + jnp.dot(qsum[:, 16:], w2_ref[1] - w2_ref[0], precision=_HIGH))
    logits = sums2 / cnt_ref[...] + r2_ref[...]
    m = jnp.max(logits, axis=1, keepdims=True)
    lse = jnp.log(jnp.sum(jnp.exp(logits - m), axis=1, keepdims=True)) + m
    o_ref[...] = logits - lse


def kernel(x, edge_index, edge_attr, w1, root1, bias1, w2, root2, bias2):
    n, f_in = x.shape
    hid = w1.shape[2]
    ncls = w2.shape[2]

    grid = (n // BR,)
    u1, xr1 = pl.pallas_call(
        _tc_pre_body,
        grid=grid,
        in_specs=[
            pl.BlockSpec((BR, f_in), lambda i: (i, 0)),
            pl.BlockSpec((2, f_in, hid), lambda i: (0, 0, 0)),
            pl.BlockSpec((f_in, hid), lambda i: (0, 0)),
        ],
        out_specs=(
            pl.BlockSpec((BR, 2 * hid), lambda i: (i, 0)),
            pl.BlockSpec((BR, hid), lambda i: (i, 0)),
        ),
        out_shape=(
            jax.ShapeDtypeStruct((n, 2 * hid), jnp.float32),
            jax.ShapeDtypeStruct((n, hid), jnp.float32),
        ),
    )(x, w1, root1)

    t1d = edge_attr.reshape(-1)
    p1 = _make_sc_scatter(1, SPLIT_L1)(u1, edge_index, t1d)

    h, r2, cnt = pl.pallas_call(
        _tc_mid_body,
        grid=grid,
        in_specs=[
            pl.BlockSpec((2, BR, ACC_W), lambda i: (0, i, 0)),
            pl.BlockSpec((BR, hid), lambda i: (i, 0)),
            pl.BlockSpec((hid,), lambda i: (0,)),
            pl.BlockSpec((hid, ncls), lambda i: (0, 0)),
            pl.BlockSpec((ncls,), lambda i: (0,)),
        ],
        out_specs=(
            pl.BlockSpec((BR, hid), lambda i: (i, 0)),
            pl.BlockSpec((BR, ncls), lambda i: (i, 0)),
            pl.BlockSpec((BR, 1), lambda i: (i, 0)),
        ),
        out_shape=(
            jax.ShapeDtypeStruct((n, hid), jnp.float32),
            jax.ShapeDtypeStruct((n, ncls), jnp.float32),
            jax.ShapeDtypeStruct((n, 1), jnp.float32),
        ),
    )(p1, xr1, bias1, root2, bias2)

    p2 = _make_sc_scatter(2, SPLIT_L2)(h, edge_index, t1d)

    out = pl.pallas_call(
        _tc_post_body,
        grid=grid,
        in_specs=[
            pl.BlockSpec((2, BR, ACC_W), lambda i: (0, i, 0)),
            pl.BlockSpec((2, hid, ncls), lambda i: (0, 0, 0)),
            pl.BlockSpec((BR, 1), lambda i: (i, 0)),
            pl.BlockSpec((BR, ncls), lambda i: (i, 0)),
        ],
        out_specs=pl.BlockSpec((BR, ncls), lambda i: (i, 0)),
        out_shape=jax.ShapeDtypeStruct((n, ncls), jnp.float32),
    )(p2, w2, cnt, r2)
    return out


# speedup vs baseline: 15.1180x; 1.1768x over previous
import functools

import jax
import jax.numpy as jnp
from jax import lax
from jax.experimental import pallas as pl
from jax.experimental.pallas import tpu as pltpu
from jax.experimental.pallas import tpu_sc as plsc

_HIGH = lax.Precision.HIGHEST

N_NODES = 10000
N_EDGES = 320000
NC = 2
NS = 16
CHUNK = 128
G_CHUNKS = N_EDGES // CHUNK
N_PAD = 10240
RPT = N_PAD // NS
ZROWS = 128
ACC_W = 32

SPLIT_L1 = (82, 74)
SPLIT_L2 = (80, 76)


def _sc_scatter_body(mode, a, b,
                     u_hbm, ei_hbm, ea_hbm, out_hbm,
                     src_v, dst_v, t_v, rows_v, msg_v, zb_v, acc_sh,
                     gsem0, gsem1, ssem0, ssem1):
    cid = lax.axis_index("c")
    sid = lax.axis_index("s")
    zvec = jnp.zeros((16,), jnp.float32)
    gsem = (gsem0, gsem1)
    ssem = (ssem0, ssem1)
    a2 = a + 2
    start0 = sid * a + 2 * jnp.minimum(sid, 2)
    start1 = NS * a + 4 + sid * b
    start = jnp.where(cid == 0, start0, start1)
    npairs = jnp.where(cid == 0, a // 2 + (sid < 2), b // 2)
    ncnt = 2 * npairs

    def _zero_zb(r, c):
        zb_v[r, pl.ds(0, 16)] = zvec
        zb_v[r, pl.ds(16, 16)] = zvec
        return c
    lax.fori_loop(0, ZROWS, _zero_zb, 0)
    for r in range(RPT // ZROWS):
        pltpu.sync_copy(zb_v, acc_sh.at[pl.ds(sid * RPT + r * ZROWS, ZROWS)])

    e0 = start * CHUNK

    @pl.when(cid == 0)
    def _stage_a():
        pltpu.sync_copy(ei_hbm.at[pl.ds(0, 1), pl.ds(e0, a2 * CHUNK)],
                        src_v.at[pl.ds(0, 1), pl.ds(0, a2 * CHUNK)])
        pltpu.sync_copy(ei_hbm.at[pl.ds(1, 1), pl.ds(e0, a2 * CHUNK)],
                        dst_v.at[pl.ds(0, 1), pl.ds(0, a2 * CHUNK)])
        pltpu.sync_copy(ea_hbm.at[pl.ds(e0, a2 * CHUNK)],
                        t_v.at[pl.ds(0, a2 * CHUNK)])

    @pl.when(cid == 1)
    def _stage_b():
        pltpu.sync_copy(ei_hbm.at[pl.ds(0, 1), pl.ds(e0, b * CHUNK)],
                        src_v.at[pl.ds(0, 1), pl.ds(0, b * CHUNK)])
        pltpu.sync_copy(ei_hbm.at[pl.ds(1, 1), pl.ds(e0, b * CHUNK)],
                        dst_v.at[pl.ds(0, 1), pl.ds(0, b * CHUNK)])
        pltpu.sync_copy(ea_hbm.at[pl.ds(e0, b * CHUNK)],
                        t_v.at[pl.ds(0, b * CHUNK)])

    if mode == 1:
        cnt_vec = (lax.iota(jnp.int32, 16) == 0).astype(jnp.float32)
        for s in range(2):
            for e in range(CHUNK):
                msg_v[s, e, pl.ds(16, 16)] = cnt_vec
    plsc.subcore_barrier()

    pltpu.async_copy(u_hbm.at[src_v.at[0, pl.ds(0, CHUNK)]], rows_v.at[0],
                     gsem[0]).wait()

    def _pair(gg, c):
        for s in range(2):
            g = 2 * gg + s
            nxt = 1 - s

            @pl.when(g + 1 < ncnt)
            def _issue():
                pltpu.async_copy(
                    u_hbm.at[src_v.at[0, pl.ds((g + 1) * CHUNK, CHUNK)]],
                    rows_v.at[nxt], gsem[nxt])

            @pl.when(g >= 2)
            def _drain_prev():
                pltpu.make_async_copy(
                    msg_v.at[s],
                    acc_sh.at[dst_v.at[0, pl.ds(g * CHUNK, CHUNK)]],
                    ssem[s]).wait()

            for jg in range(CHUNK // 16):
                tv = t_v[pl.ds(g * CHUNK + jg * 16, 16)]
                for j in range(16):
                    e = jg * 16 + j
                    if mode == 1:
                        z0 = rows_v[s, e, pl.ds(0, 16)]
                        z1 = rows_v[s, e, pl.ds(16, 16)]
                        msg_v[s, e, pl.ds(0, 16)] = z0 + tv[j] * z1
                    else:
                        hrow = rows_v[s, e, pl.ds(0, 16)]
                        msg_v[s, e, pl.ds(0, 16)] = hrow
                        msg_v[s, e, pl.ds(16, 16)] = tv[j] * hrow
            pltpu.async_copy(msg_v.at[s],
                             acc_sh.at[dst_v.at[0, pl.ds(g * CHUNK, CHUNK)]],
                             ssem[s], add=True)

            @pl.when(g + 1 < ncnt)
            def _wait_gather():
                pltpu.make_async_copy(
                    u_hbm.at[src_v.at[0, pl.ds((g + 1) * CHUNK, CHUNK)]],
                    rows_v.at[nxt], gsem[nxt]).wait()
        return c
    lax.fori_loop(0, npairs, _pair, 0)

    for s in range(2):
        pltpu.make_async_copy(msg_v.at[s],
                              acc_sh.at[dst_v.at[0, pl.ds(0, CHUNK)]],
                              ssem[s]).wait()

    plsc.subcore_barrier()
    pltpu.sync_copy(acc_sh.at[pl.ds(sid * RPT, RPT)],
                    out_hbm.at[cid, pl.ds(sid * RPT, RPT)])


def _make_sc_scatter(mode, split):
    a, b = split
    ecap = (a + 2) * CHUNK
    row_w = 32
    mesh = plsc.VectorSubcoreMesh(core_axis_name="c", subcore_axis_name="s")
    return pl.kernel(
        functools.partial(_sc_scatter_body, mode, a, b),
        out_type=jax.ShapeDtypeStruct((NC, N_PAD, ACC_W), jnp.float32),
        mesh=mesh,
        scratch_types=[
            pltpu.VMEM((1, ecap), jnp.int32),
            pltpu.VMEM((1, ecap), jnp.int32),
            pltpu.VMEM((ecap,), jnp.float32),
            pltpu.VMEM((2, CHUNK, row_w), jnp.float32),
            pltpu.VMEM((2, CHUNK, ACC_W), jnp.float32),
            pltpu.VMEM((ZROWS, ACC_W), jnp.float32),
            pltpu.VMEM_SHARED((N_PAD, ACC_W), jnp.float32),
            pltpu.SemaphoreType.DMA,
            pltpu.SemaphoreType.DMA,
            pltpu.SemaphoreType.DMA,
            pltpu.SemaphoreType.DMA,
        ],
        compiler_params=pltpu.CompilerParams(
            use_tc_tiling_on_sc=False, needs_layout_passes=False),
    )


BRP = 320


def _bd4(m):
    z = jnp.zeros_like(m)
    rows = [jnp.concatenate([m if j == i else z for j in range(4)], axis=1)
            for i in range(4)]
    return jnp.concatenate(rows, axis=0)


def _tc_pre_body(x4_ref, w1_ref, root1_ref, u_ref, xr_ref):
    w_cat = jnp.concatenate([w1_ref[0], w1_ref[1] - w1_ref[0]], axis=1)
    r1pad = jnp.concatenate(
        [root1_ref[...], jnp.zeros_like(root1_ref[...])], axis=1)
    x4 = x4_ref[...]
    u_ref[...] = jnp.dot(x4, _bd4(w_cat), precision=_HIGH)
    xr_ref[...] = jnp.dot(x4, _bd4(r1pad), precision=_HIGH)


def _tc_mid_body(p_ref, xr_ref, bias1_ref, root2_ref, bias2_ref,
                 h_ref, r2_ref, rc_ref):
    psum = p_ref[0] + p_ref[1]
    r_i = lax.broadcasted_iota(jnp.int32, (128, 128), 0)
    c_i = lax.broadcasted_iota(jnp.int32, (128, 128), 1)
    mdiv = (r_i == (c_i // 32) * 32 + 16).astype(jnp.float32)
    divisor = jnp.maximum(jnp.dot(psum, mdiv, precision=_HIGH), 1.0)
    rc_ref[...] = 1.0 / divisor
    b1 = jnp.tile(jnp.concatenate([bias1_ref[...],
                                   jnp.zeros((16,), jnp.float32)]), 4)
    pre = psum / divisor + xr_ref[...] + b1
    h = jnp.where(pre > 0, pre, jnp.exp(jnp.minimum(pre, 0.0)) - 1.0)
    h_ref[...] = h
    r2pad = jnp.concatenate(
        [root2_ref[...], jnp.zeros_like(root2_ref[...])], axis=0)
    b2 = jnp.tile(bias2_ref[...], 4)
    r2_ref[...] = jnp.dot(h, _bd4(r2pad), precision=_HIGH) + b2


def _tc_post_body(q_ref, w2_ref, rc_ref, r2_ref, o_ref):
    qsum = q_ref[0] + q_ref[1]
    w2cat = jnp.concatenate([w2_ref[0], w2_ref[1] - w2_ref[0]], axis=0)
    sums2 = jnp.dot(qsum, _bd4(w2cat), precision=_HIGH)
    logits_p = sums2 * rc_ref[...] + r2_ref[...]
    m = jnp.max(logits_p, axis=1, keepdims=True)
    ex = jnp.exp(logits_p - m)
    r_i = lax.broadcasted_iota(jnp.int32, (128, 128), 0)
    c_i = lax.broadcasted_iota(jnp.int32, (128, 128), 1)
    msum = (r_i // 32 == c_i // 32).astype(jnp.float32)
    gsum = jnp.dot(ex, msum, precision=_HIGH)
    o_ref[...] = logits_p - m - jnp.log(gsum)


def kernel(x, edge_index, edge_attr, w1, root1, bias1, w2, root2, bias2):
    n, f_in = x.shape
    hid = w1.shape[2]
    ncls = w2.shape[2]
    npad4 = N_PAD // 4
    t1d = edge_attr.reshape(-1)
    x4 = jnp.pad(x.reshape(n // 4, 4 * f_in),
                 ((0, npad4 - n // 4), (0, 0)))

    grid = (npad4 // BRP,)
    u1p, xrp = pl.pallas_call(
        _tc_pre_body,
        grid=grid,
        in_specs=[
            pl.BlockSpec((BRP, 4 * f_in), lambda i: (i, 0)),
            pl.BlockSpec((2, f_in, hid), lambda i: (0, 0, 0)),
            pl.BlockSpec((f_in, hid), lambda i: (0, 0)),
        ],
        out_specs=(
            pl.BlockSpec((BRP, 128), lambda i: (i, 0)),
            pl.BlockSpec((BRP, 128), lambda i: (i, 0)),
        ),
        out_shape=(
            jax.ShapeDtypeStruct((npad4, 128), jnp.float32),
            jax.ShapeDtypeStruct((npad4, 128), jnp.float32),
        ),
    )(x4, w1, root1)

    p1 = _make_sc_scatter(1, SPLIT_L1)(u1p.reshape(N_PAD, 32),
                                       edge_index, t1d)

    hp, r2p, rcp = pl.pallas_call(
        _tc_mid_body,
        grid=grid,
        in_specs=[
            pl.BlockSpec((2, BRP, 128), lambda i: (0, i, 0)),
            pl.BlockSpec((BRP, 128), lambda i: (i, 0)),
            pl.BlockSpec((hid,), lambda i: (0,)),
            pl.BlockSpec((hid, ncls), lambda i: (0, 0)),
            pl.BlockSpec((ncls,), lambda i: (0,)),
        ],
        out_specs=(
            pl.BlockSpec((BRP, 128), lambda i: (i, 0)),
            pl.BlockSpec((BRP, 128), lambda i: (i, 0)),
            pl.BlockSpec((BRP, 128), lambda i: (i, 0)),
        ),
        out_shape=(
            jax.ShapeDtypeStruct((npad4, 128), jnp.float32),
            jax.ShapeDtypeStruct((npad4, 128), jnp.float32),
            jax.ShapeDtypeStruct((npad4, 128), jnp.float32),
        ),
    )(p1.reshape(NC, npad4, 128), xrp, bias1, root2, bias2)

    p2 = _make_sc_scatter(2, SPLIT_L2)(hp.reshape(N_PAD, 32),
                                       edge_index, t1d)

    outp = pl.pallas_call(
        _tc_post_body,
        grid=grid,
        in_specs=[
            pl.BlockSpec((2, BRP, 128), lambda i: (0, i, 0)),
            pl.BlockSpec((2, hid, ncls), lambda i: (0, 0, 0)),
            pl.BlockSpec((BRP, 128), lambda i: (i, 0)),
            pl.BlockSpec((BRP, 128), lambda i: (i, 0)),
        ],
        out_specs=pl.BlockSpec((BRP, 128), lambda i: (i, 0)),
        out_shape=jax.ShapeDtypeStruct((npad4, 128), jnp.float32),
    )(p2.reshape(NC, npad4, 128), w2, rcp, r2p)
    return outp.reshape(N_PAD, ncls)[:n]


# speedup vs baseline: 15.1452x; 1.0018x over previous
import functools

import jax
import jax.numpy as jnp
from jax import lax
from jax.experimental import pallas as pl
from jax.experimental.pallas import tpu as pltpu
from jax.experimental.pallas import tpu_sc as plsc

_HIGH = lax.Precision.HIGHEST

N_NODES = 10000
N_EDGES = 320000
NC = 2
NS = 16
CHUNK = 128
G_CHUNKS = N_EDGES // CHUNK
N_PAD = 10240
RPT = N_PAD // NS
ZROWS = 128
ACC_W = 32

SPLIT_L1 = (82, 74)
SPLIT_L2 = (80, 76)


def _sc_scatter_body(mode, a, b,
                     u_hbm, ei_hbm, ea_hbm, out_hbm,
                     src_v, dst_v, t_v, rows_v, msg_v, zb_v, acc_sh,
                     gsem0, gsem1, ssem0, ssem1):
    cid = lax.axis_index("c")
    sid = lax.axis_index("s")
    zvec = jnp.zeros((16,), jnp.float32)
    gsem = (gsem0, gsem1)
    ssem = (ssem0, ssem1)
    a2 = a + 2
    start0 = sid * a + 2 * jnp.minimum(sid, 2)
    start1 = NS * a + 4 + sid * b
    start = jnp.where(cid == 0, start0, start1)
    npairs = jnp.where(cid == 0, a // 2 + (sid < 2), b // 2)
    ncnt = 2 * npairs

    def _zero_zb(r, c):
        zb_v[r, pl.ds(0, 16)] = zvec
        zb_v[r, pl.ds(16, 16)] = zvec
        return c
    lax.fori_loop(0, ZROWS, _zero_zb, 0)
    for r in range(RPT // ZROWS):
        pltpu.sync_copy(zb_v, acc_sh.at[pl.ds(sid * RPT + r * ZROWS, ZROWS)])

    e0 = start * CHUNK

    @pl.when(cid == 0)
    def _stage_a():
        pltpu.sync_copy(ei_hbm.at[pl.ds(0, 1), pl.ds(e0, a2 * CHUNK)],
                        src_v.at[pl.ds(0, 1), pl.ds(0, a2 * CHUNK)])
        pltpu.sync_copy(ei_hbm.at[pl.ds(1, 1), pl.ds(e0, a2 * CHUNK)],
                        dst_v.at[pl.ds(0, 1), pl.ds(0, a2 * CHUNK)])
        pltpu.sync_copy(ea_hbm.at[pl.ds(e0, a2 * CHUNK)],
                        t_v.at[pl.ds(0, a2 * CHUNK)])

    @pl.when(cid == 1)
    def _stage_b():
        pltpu.sync_copy(ei_hbm.at[pl.ds(0, 1), pl.ds(e0, b * CHUNK)],
                        src_v.at[pl.ds(0, 1), pl.ds(0, b * CHUNK)])
        pltpu.sync_copy(ei_hbm.at[pl.ds(1, 1), pl.ds(e0, b * CHUNK)],
                        dst_v.at[pl.ds(0, 1), pl.ds(0, b * CHUNK)])
        pltpu.sync_copy(ea_hbm.at[pl.ds(e0, b * CHUNK)],
                        t_v.at[pl.ds(0, b * CHUNK)])

    if mode == 1:
        cnt_vec = (lax.iota(jnp.int32, 16) == 0).astype(jnp.float32)
        for s in range(2):
            for e in range(CHUNK):
                msg_v[s, e, pl.ds(16, 16)] = cnt_vec
    plsc.subcore_barrier()

    pltpu.async_copy(u_hbm.at[src_v.at[0, pl.ds(0, CHUNK)]], rows_v.at[0],
                     gsem[0]).wait()

    def _pair(gg, c):
        for s in range(2):
            g = 2 * gg + s
            nxt = 1 - s

            @pl.when(g + 1 < ncnt)
            def _issue():
                pltpu.async_copy(
                    u_hbm.at[src_v.at[0, pl.ds((g + 1) * CHUNK, CHUNK)]],
                    rows_v.at[nxt], gsem[nxt])

            @pl.when(g >= 2)
            def _drain_prev():
                pltpu.make_async_copy(
                    msg_v.at[s],
                    acc_sh.at[dst_v.at[0, pl.ds(g * CHUNK, CHUNK)]],
                    ssem[s]).wait()

            for jg in range(CHUNK // 16):
                tv = t_v[pl.ds(g * CHUNK + jg * 16, 16)]
                for j in range(16):
                    e = jg * 16 + j
                    if mode == 1:
                        z0 = rows_v[s, e, pl.ds(0, 16)]
                        z1 = rows_v[s, e, pl.ds(16, 16)]
                        msg_v[s, e, pl.ds(0, 16)] = z0 + tv[j] * z1
                    else:
                        hrow = rows_v[s, e, pl.ds(0, 16)]
                        msg_v[s, e, pl.ds(0, 16)] = hrow
                        msg_v[s, e, pl.ds(16, 16)] = tv[j] * hrow
            pltpu.async_copy(msg_v.at[s],
                             acc_sh.at[dst_v.at[0, pl.ds(g * CHUNK, CHUNK)]],
                             ssem[s], add=True)

            @pl.when(g + 1 < ncnt)
            def _wait_gather():
                pltpu.make_async_copy(
                    u_hbm.at[src_v.at[0, pl.ds((g + 1) * CHUNK, CHUNK)]],
                    rows_v.at[nxt], gsem[nxt]).wait()
        return c
    lax.fori_loop(0, npairs, _pair, 0)

    for s in range(2):
        pltpu.make_async_copy(msg_v.at[s],
                              acc_sh.at[dst_v.at[0, pl.ds(0, CHUNK)]],
                              ssem[s]).wait()

    plsc.subcore_barrier()
    pltpu.sync_copy(acc_sh.at[pl.ds(sid * RPT, RPT)],
                    out_hbm.at[cid, pl.ds(sid * RPT, RPT)])


def _make_sc_scatter(mode, split):
    a, b = split
    ecap = (a + 2) * CHUNK
    row_w = 32
    mesh = plsc.VectorSubcoreMesh(core_axis_name="c", subcore_axis_name="s")
    return pl.kernel(
        functools.partial(_sc_scatter_body, mode, a, b),
        out_type=jax.ShapeDtypeStruct((NC, N_PAD, ACC_W), jnp.float32),
        mesh=mesh,
        scratch_types=[
            pltpu.VMEM((1, ecap), jnp.int32),
            pltpu.VMEM((1, ecap), jnp.int32),
            pltpu.VMEM((ecap,), jnp.float32),
            pltpu.VMEM((2, CHUNK, row_w), jnp.float32),
            pltpu.VMEM((2, CHUNK, ACC_W), jnp.float32),
            pltpu.VMEM((ZROWS, ACC_W), jnp.float32),
            pltpu.VMEM_SHARED((N_PAD, ACC_W), jnp.float32),
            pltpu.SemaphoreType.DMA,
            pltpu.SemaphoreType.DMA,
            pltpu.SemaphoreType.DMA,
            pltpu.SemaphoreType.DMA,
        ],
        compiler_params=pltpu.CompilerParams(
            use_tc_tiling_on_sc=False, needs_layout_passes=False,
            disable_bounds_checks=True),
    )


BRP = 320


def _bd4(m):
    z = jnp.zeros_like(m)
    rows = [jnp.concatenate([m if j == i else z for j in range(4)], axis=1)
            for i in range(4)]
    return jnp.concatenate(rows, axis=0)


def _tc_pre_body(x4_ref, w1_ref, root1_ref, u_ref, xr_ref):
    w_cat = jnp.concatenate([w1_ref[0], w1_ref[1] - w1_ref[0]], axis=1)
    r1pad = jnp.concatenate(
        [root1_ref[...], jnp.zeros_like(root1_ref[...])], axis=1)
    x4 = x4_ref[...]
    u_ref[...] = jnp.dot(x4, _bd4(w_cat), precision=_HIGH)
    xr_ref[...] = jnp.dot(x4, _bd4(r1pad), precision=_HIGH)


def _tc_mid_body(p_ref, xr_ref, bias1_ref, root2_ref, bias2_ref,
                 h_ref, r2_ref, rc_ref):
    psum = p_ref[0] + p_ref[1]
    r_i = lax.broadcasted_iota(jnp.int32, (128, 128), 0)
    c_i = lax.broadcasted_iota(jnp.int32, (128, 128), 1)
    mdiv = (r_i == (c_i // 32) * 32 + 16).astype(jnp.float32)
    divisor = jnp.maximum(jnp.dot(psum, mdiv, precision=_HIGH), 1.0)
    rc_ref[...] = 1.0 / divisor
    b1 = jnp.tile(jnp.concatenate([bias1_ref[...],
                                   jnp.zeros((16,), jnp.float32)]), 4)
    pre = psum / divisor + xr_ref[...] + b1
    h = jnp.where(pre > 0, pre, jnp.exp(jnp.minimum(pre, 0.0)) - 1.0)
    h_ref[...] = h
    r2pad = jnp.concatenate(
        [root2_ref[...], jnp.zeros_like(root2_ref[...])], axis=0)
    b2 = jnp.tile(bias2_ref[...], 4)
    r2_ref[...] = jnp.dot(h, _bd4(r2pad), precision=_HIGH) + b2


def _tc_post_body(q_ref, w2_ref, rc_ref, r2_ref, o_ref):
    qsum = q_ref[0] + q_ref[1]
    w2cat = jnp.concatenate([w2_ref[0], w2_ref[1] - w2_ref[0]], axis=0)
    sums2 = jnp.dot(qsum, _bd4(w2cat), precision=_HIGH)
    logits_p = sums2 * rc_ref[...] + r2_ref[...]
    m = jnp.max(logits_p, axis=1, keepdims=True)
    ex = jnp.exp(logits_p - m)
    r_i = lax.broadcasted_iota(jnp.int32, (128, 128), 0)
    c_i = lax.broadcasted_iota(jnp.int32, (128, 128), 1)
    msum = (r_i // 32 == c_i // 32).astype(jnp.float32)
    gsum = jnp.dot(ex, msum, precision=_HIGH)
    o_ref[...] = logits_p - m - jnp.log(gsum)


def kernel(x, edge_index, edge_attr, w1, root1, bias1, w2, root2, bias2):
    n, f_in = x.shape
    hid = w1.shape[2]
    ncls = w2.shape[2]
    npad4 = N_PAD // 4
    t1d = edge_attr.reshape(-1)
    x4 = jnp.pad(x.reshape(n // 4, 4 * f_in),
                 ((0, npad4 - n // 4), (0, 0)))

    grid = (npad4 // BRP,)
    u1p, xrp = pl.pallas_call(
        _tc_pre_body,
        grid=grid,
        in_specs=[
            pl.BlockSpec((BRP, 4 * f_in), lambda i: (i, 0)),
            pl.BlockSpec((2, f_in, hid), lambda i: (0, 0, 0)),
            pl.BlockSpec((f_in, hid), lambda i: (0, 0)),
        ],
        out_specs=(
            pl.BlockSpec((BRP, 128), lambda i: (i, 0)),
            pl.BlockSpec((BRP, 128), lambda i: (i, 0)),
        ),
        out_shape=(
            jax.ShapeDtypeStruct((npad4, 128), jnp.float32),
            jax.ShapeDtypeStruct((npad4, 128), jnp.float32),
        ),
    )(x4, w1, root1)

    p1 = _make_sc_scatter(1, SPLIT_L1)(u1p.reshape(N_PAD, 32),
                                       edge_index, t1d)

    hp, r2p, rcp = pl.pallas_call(
        _tc_mid_body,
        grid=grid,
        in_specs=[
            pl.BlockSpec((2, BRP, 128), lambda i: (0, i, 0)),
            pl.BlockSpec((BRP, 128), lambda i: (i, 0)),
            pl.BlockSpec((hid,), lambda i: (0,)),
            pl.BlockSpec((hid, ncls), lambda i: (0, 0)),
            pl.BlockSpec((ncls,), lambda i: (0,)),
        ],
        out_specs=(
            pl.BlockSpec((BRP, 128), lambda i: (i, 0)),
            pl.BlockSpec((BRP, 128), lambda i: (i, 0)),
            pl.BlockSpec((BRP, 128), lambda i: (i, 0)),
        ),
        out_shape=(
            jax.ShapeDtypeStruct((npad4, 128), jnp.float32),
            jax.ShapeDtypeStruct((npad4, 128), jnp.float32),
            jax.ShapeDtypeStruct((npad4, 128), jnp.float32),
        ),
    )(p1.reshape(NC, npad4, 128), xrp, bias1, root2, bias2)

    p2 = _make_sc_scatter(2, SPLIT_L2)(hp.reshape(N_PAD, 32),
                                       edge_index, t1d)

    outp = pl.pallas_call(
        _tc_post_body,
        grid=grid,
        in_specs=[
            pl.BlockSpec((2, BRP, 128), lambda i: (0, i, 0)),
            pl.BlockSpec((2, hid, ncls), lambda i: (0, 0, 0)),
            pl.BlockSpec((BRP, 128), lambda i: (i, 0)),
            pl.BlockSpec((BRP, 128), lambda i: (i, 0)),
        ],
        out_specs=pl.BlockSpec((BRP, 128), lambda i: (i, 0)),
        out_shape=jax.ShapeDtypeStruct((npad4, 128), jnp.float32),
    )(p2.reshape(NC, npad4, 128), w2, rcp, r2p)
    return outp.reshape(N_PAD, ncls)[:n]


# speedup vs baseline: 20.3534x; 1.3439x over previous
import functools

import jax
import jax.numpy as jnp
from jax import lax
from jax.experimental import pallas as pl
from jax.experimental.pallas import tpu as pltpu
from jax.experimental.pallas import tpu_sc as plsc

_HIGH = lax.Precision.HIGHEST

N_NODES = 10000
N_EDGES = 320000
NC = 2
NS = 16
CHUNK = 128
G_CHUNKS = N_EDGES // CHUNK
N_PAD = 10240
RPT = N_PAD // NS
ZROWS = 128
ACC_W = 32

SPLIT_L1 = (76, 80)
SPLIT_L2 = (76, 80)


def _sc_scatter_body(mode, a, b,
                     u_hbm, ei_hbm, ea_hbm, out_hbm,
                     src_v, dst_v, t_v, rows_v, msg_v, zb_v, acc_sh,
                     gsem0, gsem1, gsem2, gsem3, ssem0, ssem1):
    cid = lax.axis_index("c")
    sid = lax.axis_index("s")
    zvec = jnp.zeros((16,), jnp.float32)
    gsem = (gsem0, gsem1, gsem2, gsem3)
    ssem = (ssem0, ssem1)
    a4 = a + 4
    start0 = sid * a + 4 * jnp.minimum(sid, 1)
    start1 = NS * a + 4 + sid * b
    start = jnp.where(cid == 0, start0, start1)
    nquads = jnp.where(cid == 0, a // 4 + (sid < 1), b // 4)
    ncnt = 4 * nquads

    def _zero_zb(r, c):
        zb_v[r, pl.ds(0, 16)] = zvec
        zb_v[r, pl.ds(16, 16)] = zvec
        return c
    lax.fori_loop(0, ZROWS, _zero_zb, 0)
    for r in range(RPT // ZROWS):
        pltpu.sync_copy(zb_v, acc_sh.at[pl.ds(sid * RPT + r * ZROWS, ZROWS)])

    e0 = start * CHUNK

    @pl.when(cid == 0)
    def _stage_a():
        pltpu.sync_copy(ei_hbm.at[pl.ds(0, 1), pl.ds(e0, a4 * CHUNK)],
                        src_v.at[pl.ds(0, 1), pl.ds(0, a4 * CHUNK)])
        pltpu.sync_copy(ei_hbm.at[pl.ds(1, 1), pl.ds(e0, a4 * CHUNK)],
                        dst_v.at[pl.ds(0, 1), pl.ds(0, a4 * CHUNK)])
        pltpu.sync_copy(ea_hbm.at[pl.ds(e0, a4 * CHUNK)],
                        t_v.at[pl.ds(0, a4 * CHUNK)])

    @pl.when(cid == 1)
    def _stage_b():
        pltpu.sync_copy(ei_hbm.at[pl.ds(0, 1), pl.ds(e0, b * CHUNK)],
                        src_v.at[pl.ds(0, 1), pl.ds(0, b * CHUNK)])
        pltpu.sync_copy(ei_hbm.at[pl.ds(1, 1), pl.ds(e0, b * CHUNK)],
                        dst_v.at[pl.ds(0, 1), pl.ds(0, b * CHUNK)])
        pltpu.sync_copy(ea_hbm.at[pl.ds(e0, b * CHUNK)],
                        t_v.at[pl.ds(0, b * CHUNK)])

    if mode == 1:
        cnt_vec = (lax.iota(jnp.int32, 16) == 0).astype(jnp.float32)
        for s in range(2):
            for e in range(CHUNK):
                msg_v[s, e, pl.ds(16, 16)] = cnt_vec
    plsc.subcore_barrier()

    pltpu.async_copy(u_hbm.at[src_v.at[0, pl.ds(0, CHUNK)]], rows_v.at[0],
                     gsem[0])

    @pl.when(1 < ncnt)
    def _prime1():
        pltpu.async_copy(u_hbm.at[src_v.at[0, pl.ds(CHUNK, CHUNK)]],
                         rows_v.at[1], gsem[1])

    def _quad(gg, c):
        for s in range(4):
            g = 4 * gg + s
            ms = s % 2
            nx2 = (s + 2) % 4

            pltpu.make_async_copy(
                u_hbm.at[src_v.at[0, pl.ds(g * CHUNK, CHUNK)]],
                rows_v.at[s], gsem[s]).wait()

            @pl.when(g + 2 < ncnt)
            def _issue():
                pltpu.async_copy(
                    u_hbm.at[src_v.at[0, pl.ds((g + 2) * CHUNK, CHUNK)]],
                    rows_v.at[nx2], gsem[nx2])

            @pl.when(g >= 2)
            def _drain_prev():
                pltpu.make_async_copy(
                    msg_v.at[ms],
                    acc_sh.at[dst_v.at[0, pl.ds(g * CHUNK, CHUNK)]],
                    ssem[ms]).wait()

            for jg in range(CHUNK // 16):
                tv = t_v[pl.ds(g * CHUNK + jg * 16, 16)]
                for j in range(16):
                    e = jg * 16 + j
                    if mode == 1:
                        z0 = rows_v[s, e, pl.ds(0, 16)]
                        z1 = rows_v[s, e, pl.ds(16, 16)]
                        msg_v[ms, e, pl.ds(0, 16)] = z0 + tv[j] * z1
                    else:
                        hrow = rows_v[s, e, pl.ds(0, 16)]
                        msg_v[ms, e, pl.ds(0, 16)] = hrow
                        msg_v[ms, e, pl.ds(16, 16)] = tv[j] * hrow
            pltpu.async_copy(msg_v.at[ms],
                             acc_sh.at[dst_v.at[0, pl.ds(g * CHUNK, CHUNK)]],
                             ssem[ms], add=True)
        return c
    lax.fori_loop(0, nquads, _quad, 0)

    for s in range(2):
        pltpu.make_async_copy(msg_v.at[s],
                              acc_sh.at[dst_v.at[0, pl.ds(0, CHUNK)]],
                              ssem[s]).wait()

    plsc.subcore_barrier()
    pltpu.sync_copy(acc_sh.at[pl.ds(sid * RPT, RPT)],
                    out_hbm.at[cid, pl.ds(sid * RPT, RPT)])


def _make_sc_scatter(mode, split):
    a, b = split
    ecap = (a + 4) * CHUNK
    row_w = 32
    mesh = plsc.VectorSubcoreMesh(core_axis_name="c", subcore_axis_name="s")
    return pl.kernel(
        functools.partial(_sc_scatter_body, mode, a, b),
        out_type=jax.ShapeDtypeStruct((NC, N_PAD, ACC_W), jnp.float32),
        mesh=mesh,
        scratch_types=[
            pltpu.VMEM((1, ecap), jnp.int32),
            pltpu.VMEM((1, ecap), jnp.int32),
            pltpu.VMEM((ecap,), jnp.float32),
            pltpu.VMEM((4, CHUNK, row_w), jnp.float32),
            pltpu.VMEM((2, CHUNK, ACC_W), jnp.float32),
            pltpu.VMEM((ZROWS, ACC_W), jnp.float32),
            pltpu.VMEM_SHARED((N_PAD, ACC_W), jnp.float32),
            pltpu.SemaphoreType.DMA,
            pltpu.SemaphoreType.DMA,
            pltpu.SemaphoreType.DMA,
            pltpu.SemaphoreType.DMA,
            pltpu.SemaphoreType.DMA,
            pltpu.SemaphoreType.DMA,
        ],
        compiler_params=pltpu.CompilerParams(
            use_tc_tiling_on_sc=False, needs_layout_passes=False,
            disable_bounds_checks=True),
    )


BRP = 320


def _bd4(m):
    z = jnp.zeros_like(m)
    rows = [jnp.concatenate([m if j == i else z for j in range(4)], axis=1)
            for i in range(4)]
    return jnp.concatenate(rows, axis=0)


def _tc_pre_body(x4_ref, w1_ref, root1_ref, u_ref, xr_ref):
    w_cat = jnp.concatenate([w1_ref[0], w1_ref[1] - w1_ref[0]], axis=1)
    r1pad = jnp.concatenate(
        [root1_ref[...], jnp.zeros_like(root1_ref[...])], axis=1)
    x4 = x4_ref[...]
    u_ref[...] = jnp.dot(x4, _bd4(w_cat), precision=_HIGH)
    xr_ref[...] = jnp.dot(x4, _bd4(r1pad), precision=_HIGH)


def _tc_mid_body(p_ref, xr_ref, bias1_ref, root2_ref, bias2_ref,
                 h_ref, r2_ref, rc_ref):
    psum = p_ref[0] + p_ref[1]
    r_i = lax.broadcasted_iota(jnp.int32, (128, 128), 0)
    c_i = lax.broadcasted_iota(jnp.int32, (128, 128), 1)
    mdiv = (r_i == (c_i // 32) * 32 + 16).astype(jnp.float32)
    divisor = jnp.maximum(jnp.dot(psum, mdiv, precision=_HIGH), 1.0)
    rc_ref[...] = 1.0 / divisor
    b1 = jnp.tile(jnp.concatenate([bias1_ref[...],
                                   jnp.zeros((16,), jnp.float32)]), 4)
    pre = psum / divisor + xr_ref[...] + b1
    h = jnp.where(pre > 0, pre, jnp.exp(jnp.minimum(pre, 0.0)) - 1.0)
    h_ref[...] = h
    r2pad = jnp.concatenate(
        [root2_ref[...], jnp.zeros_like(root2_ref[...])], axis=0)
    b2 = jnp.tile(bias2_ref[...], 4)
    r2_ref[...] = jnp.dot(h, _bd4(r2pad), precision=_HIGH) + b2


def _tc_post_body(q_ref, w2_ref, rc_ref, r2_ref, o_ref):
    qsum = q_ref[0] + q_ref[1]
    w2cat = jnp.concatenate([w2_ref[0], w2_ref[1] - w2_ref[0]], axis=0)
    sums2 = jnp.dot(qsum, _bd4(w2cat), precision=_HIGH)
    logits_p = sums2 * rc_ref[...] + r2_ref[...]
    m = jnp.max(logits_p, axis=1, keepdims=True)
    ex = jnp.exp(logits_p - m)
    r_i = lax.broadcasted_iota(jnp.int32, (128, 128), 0)
    c_i = lax.broadcasted_iota(jnp.int32, (128, 128), 1)
    msum = (r_i // 32 == c_i // 32).astype(jnp.float32)
    gsum = jnp.dot(ex, msum, precision=_HIGH)
    o_ref[...] = logits_p - m - jnp.log(gsum)


def kernel(x, edge_index, edge_attr, w1, root1, bias1, w2, root2, bias2):
    n, f_in = x.shape
    hid = w1.shape[2]
    ncls = w2.shape[2]
    npad4 = N_PAD // 4
    t1d = edge_attr.reshape(-1)
    x4 = jnp.pad(x.reshape(n // 4, 4 * f_in),
                 ((0, npad4 - n // 4), (0, 0)))

    grid = (npad4 // BRP,)
    u1p, xrp = pl.pallas_call(
        _tc_pre_body,
        grid=grid,
        in_specs=[
            pl.BlockSpec((BRP, 4 * f_in), lambda i: (i, 0)),
            pl.BlockSpec((2, f_in, hid), lambda i: (0, 0, 0)),
            pl.BlockSpec((f_in, hid), lambda i: (0, 0)),
        ],
        out_specs=(
            pl.BlockSpec((BRP, 128), lambda i: (i, 0)),
            pl.BlockSpec((BRP, 128), lambda i: (i, 0)),
        ),
        out_shape=(
            jax.ShapeDtypeStruct((npad4, 128), jnp.float32),
            jax.ShapeDtypeStruct((npad4, 128), jnp.float32),
        ),
    )(x4, w1, root1)

    p1 = _make_sc_scatter(1, SPLIT_L1)(u1p.reshape(N_PAD, 32),
                                       edge_index, t1d)

    hp, r2p, rcp = pl.pallas_call(
        _tc_mid_body,
        grid=grid,
        in_specs=[
            pl.BlockSpec((2, BRP, 128), lambda i: (0, i, 0)),
            pl.BlockSpec((BRP, 128), lambda i: (i, 0)),
            pl.BlockSpec((hid,), lambda i: (0,)),
            pl.BlockSpec((hid, ncls), lambda i: (0, 0)),
            pl.BlockSpec((ncls,), lambda i: (0,)),
        ],
        out_specs=(
            pl.BlockSpec((BRP, 128), lambda i: (i, 0)),
            pl.BlockSpec((BRP, 128), lambda i: (i, 0)),
            pl.BlockSpec((BRP, 128), lambda i: (i, 0)),
        ),
        out_shape=(
            jax.ShapeDtypeStruct((npad4, 128), jnp.float32),
            jax.ShapeDtypeStruct((npad4, 128), jnp.float32),
            jax.ShapeDtypeStruct((npad4, 128), jnp.float32),
        ),
    )(p1.reshape(NC, npad4, 128), xrp, bias1, root2, bias2)

    p2 = _make_sc_scatter(2, SPLIT_L2)(hp.reshape(N_PAD, 32),
                                       edge_index, t1d)

    outp = pl.pallas_call(
        _tc_post_body,
        grid=grid,
        in_specs=[
            pl.BlockSpec((2, BRP, 128), lambda i: (0, i, 0)),
            pl.BlockSpec((2, hid, ncls), lambda i: (0, 0, 0)),
            pl.BlockSpec((BRP, 128), lambda i: (i, 0)),
            pl.BlockSpec((BRP, 128), lambda i: (i, 0)),
        ],
        out_specs=pl.BlockSpec((BRP, 128), lambda i: (i, 0)),
        out_shape=jax.ShapeDtypeStruct((npad4, 128), jnp.float32),
    )(p2.reshape(NC, npad4, 128), w2, rcp, r2p)
    return outp.reshape(N_PAD, ncls)[:n]


# speedup vs baseline: 21.9893x; 1.0804x over previous
import functools

import jax
import jax.numpy as jnp
from jax import lax
from jax.experimental import pallas as pl
from jax.experimental.pallas import tpu as pltpu
from jax.experimental.pallas import tpu_sc as plsc

_HIGH = lax.Precision.HIGHEST

N_NODES = 10000
N_EDGES = 320000
NC = 2
NS = 16
CHUNK = 128
G_CHUNKS = N_EDGES // CHUNK
N_PAD = 10240
RPT = N_PAD // NS
ZROWS = 128
ACC_W = 32

SPLIT_L1 = (76, 80)
SPLIT_L2 = (76, 80)


def _sc_scatter_body(mode, a, b,
                     u_hbm, ei_hbm, ea_hbm, out_hbm,
                     src_v, dst_v, t_v, rows_v, msg_v, zb_v, acc_sh,
                     gsem0, gsem1, gsem2, gsem3, ssem0, ssem1):
    cid = lax.axis_index("c")
    sid = lax.axis_index("s")
    zvec = jnp.zeros((16,), jnp.float32)
    gsem = (gsem0, gsem1, gsem2, gsem3)
    ssem = (ssem0, ssem1)
    a4 = a + 4
    start0 = sid * a + 4 * jnp.minimum(sid, 1)
    start1 = NS * a + 4 + sid * b
    start = jnp.where(cid == 0, start0, start1)
    nquads = jnp.where(cid == 0, a // 4 + (sid < 1), b // 4)
    ncnt = 4 * nquads

    def _zero_zb(r, c):
        zb_v[r, pl.ds(0, 16)] = zvec
        zb_v[r, pl.ds(16, 16)] = zvec
        return c
    lax.fori_loop(0, ZROWS, _zero_zb, 0)
    for r in range(RPT // ZROWS):
        pltpu.sync_copy(zb_v, acc_sh.at[pl.ds(sid * RPT + r * ZROWS, ZROWS)])

    e0 = start * CHUNK

    @pl.when(cid == 0)
    def _stage_a():
        pltpu.sync_copy(ei_hbm.at[pl.ds(0, 1), pl.ds(e0, a4 * CHUNK)],
                        src_v.at[pl.ds(0, 1), pl.ds(0, a4 * CHUNK)])
        pltpu.sync_copy(ei_hbm.at[pl.ds(1, 1), pl.ds(e0, a4 * CHUNK)],
                        dst_v.at[pl.ds(0, 1), pl.ds(0, a4 * CHUNK)])
        pltpu.sync_copy(ea_hbm.at[pl.ds(e0, a4 * CHUNK)],
                        t_v.at[pl.ds(0, a4 * CHUNK)])

    @pl.when(cid == 1)
    def _stage_b():
        pltpu.sync_copy(ei_hbm.at[pl.ds(0, 1), pl.ds(e0, b * CHUNK)],
                        src_v.at[pl.ds(0, 1), pl.ds(0, b * CHUNK)])
        pltpu.sync_copy(ei_hbm.at[pl.ds(1, 1), pl.ds(e0, b * CHUNK)],
                        dst_v.at[pl.ds(0, 1), pl.ds(0, b * CHUNK)])
        pltpu.sync_copy(ea_hbm.at[pl.ds(e0, b * CHUNK)],
                        t_v.at[pl.ds(0, b * CHUNK)])

    if mode == 1:
        cnt_vec = (lax.iota(jnp.int32, 16) == 0).astype(jnp.float32)
        for s in range(2):
            for e in range(CHUNK):
                msg_v[s, e, pl.ds(16, 16)] = cnt_vec
    plsc.subcore_barrier()

    pltpu.async_copy(u_hbm.at[src_v.at[0, pl.ds(0, CHUNK)]], rows_v.at[0],
                     gsem[0])

    @pl.when(1 < ncnt)
    def _prime1():
        pltpu.async_copy(u_hbm.at[src_v.at[0, pl.ds(CHUNK, CHUNK)]],
                         rows_v.at[1], gsem[1])

    @pl.when(2 < ncnt)
    def _prime2():
        pltpu.async_copy(u_hbm.at[src_v.at[0, pl.ds(2 * CHUNK, CHUNK)]],
                         rows_v.at[2], gsem[2])

    def _quad(gg, c):
        for s in range(4):
            g = 4 * gg + s
            ms = s % 2
            nx2 = (s + 3) % 4

            pltpu.make_async_copy(
                u_hbm.at[src_v.at[0, pl.ds(g * CHUNK, CHUNK)]],
                rows_v.at[s], gsem[s]).wait()

            @pl.when(g + 3 < ncnt)
            def _issue():
                pltpu.async_copy(
                    u_hbm.at[src_v.at[0, pl.ds((g + 3) * CHUNK, CHUNK)]],
                    rows_v.at[nx2], gsem[nx2])

            @pl.when(g >= 2)
            def _drain_prev():
                pltpu.make_async_copy(
                    msg_v.at[ms],
                    acc_sh.at[dst_v.at[0, pl.ds(g * CHUNK, CHUNK)]],
                    ssem[ms]).wait()

            for jg in range(CHUNK // 16):
                tv = t_v[pl.ds(g * CHUNK + jg * 16, 16)]
                for j in range(16):
                    e = jg * 16 + j
                    if mode == 1:
                        z0 = rows_v[s, e, pl.ds(0, 16)]
                        z1 = rows_v[s, e, pl.ds(16, 16)]
                        msg_v[ms, e, pl.ds(0, 16)] = z0 + tv[j] * z1
                    else:
                        hrow = rows_v[s, e, pl.ds(0, 16)]
                        msg_v[ms, e, pl.ds(0, 16)] = hrow
                        msg_v[ms, e, pl.ds(16, 16)] = tv[j] * hrow
            pltpu.async_copy(msg_v.at[ms],
                             acc_sh.at[dst_v.at[0, pl.ds(g * CHUNK, CHUNK)]],
                             ssem[ms], add=True)
        return c
    lax.fori_loop(0, nquads, _quad, 0)

    for s in range(2):
        pltpu.make_async_copy(msg_v.at[s],
                              acc_sh.at[dst_v.at[0, pl.ds(0, CHUNK)]],
                              ssem[s]).wait()

    plsc.subcore_barrier()
    pltpu.sync_copy(acc_sh.at[pl.ds(sid * RPT, RPT)],
                    out_hbm.at[cid, pl.ds(sid * RPT, RPT)])


def _make_sc_scatter(mode, split):
    a, b = split
    ecap = (a + 4) * CHUNK
    row_w = 32
    mesh = plsc.VectorSubcoreMesh(core_axis_name="c", subcore_axis_name="s")
    return pl.kernel(
        functools.partial(_sc_scatter_body, mode, a, b),
        out_type=jax.ShapeDtypeStruct((NC, N_PAD, ACC_W), jnp.float32),
        mesh=mesh,
        scratch_types=[
            pltpu.VMEM((1, ecap), jnp.int32),
            pltpu.VMEM((1, ecap), jnp.int32),
            pltpu.VMEM((ecap,), jnp.float32),
            pltpu.VMEM((4, CHUNK, row_w), jnp.float32),
            pltpu.VMEM((2, CHUNK, ACC_W), jnp.float32),
            pltpu.VMEM((ZROWS, ACC_W), jnp.float32),
            pltpu.VMEM_SHARED((N_PAD, ACC_W), jnp.float32),
            pltpu.SemaphoreType.DMA,
            pltpu.SemaphoreType.DMA,
            pltpu.SemaphoreType.DMA,
            pltpu.SemaphoreType.DMA,
            pltpu.SemaphoreType.DMA,
            pltpu.SemaphoreType.DMA,
        ],
        compiler_params=pltpu.CompilerParams(
            use_tc_tiling_on_sc=False, needs_layout_passes=False,
            disable_bounds_checks=True),
    )


BRP = 320


def _bd4(m):
    z = jnp.zeros_like(m)
    rows = [jnp.concatenate([m if j == i else z for j in range(4)], axis=1)
            for i in range(4)]
    return jnp.concatenate(rows, axis=0)


def _tc_pre_body(x4_ref, w1_ref, root1_ref, u_ref, xr_ref):
    w_cat = jnp.concatenate([w1_ref[0], w1_ref[1] - w1_ref[0]], axis=1)
    r1pad = jnp.concatenate(
        [root1_ref[...], jnp.zeros_like(root1_ref[...])], axis=1)
    x4 = x4_ref[...]
    u_ref[...] = jnp.dot(x4, _bd4(w_cat), precision=_HIGH)
    xr_ref[...] = jnp.dot(x4, _bd4(r1pad), precision=_HIGH)


def _tc_mid_body(p_ref, xr_ref, bias1_ref, root2_ref, bias2_ref,
                 h_ref, r2_ref, rc_ref):
    psum = p_ref[0] + p_ref[1]
    r_i = lax.broadcasted_iota(jnp.int32, (128, 128), 0)
    c_i = lax.broadcasted_iota(jnp.int32, (128, 128), 1)
    mdiv = (r_i == (c_i // 32) * 32 + 16).astype(jnp.float32)
    divisor = jnp.maximum(jnp.dot(psum, mdiv, precision=_HIGH), 1.0)
    rc_ref[...] = 1.0 / divisor
    b1 = jnp.tile(jnp.concatenate([bias1_ref[...],
                                   jnp.zeros((16,), jnp.float32)]), 4)
    pre = psum / divisor + xr_ref[...] + b1
    h = jnp.where(pre > 0, pre, jnp.exp(jnp.minimum(pre, 0.0)) - 1.0)
    h_ref[...] = h
    r2pad = jnp.concatenate(
        [root2_ref[...], jnp.zeros_like(root2_ref[...])], axis=0)
    b2 = jnp.tile(bias2_ref[...], 4)
    r2_ref[...] = jnp.dot(h, _bd4(r2pad), precision=_HIGH) + b2


def _tc_post_body(q_ref, w2_ref, rc_ref, r2_ref, o_ref):
    qsum = q_ref[0] + q_ref[1]
    w2cat = jnp.concatenate([w2_ref[0], w2_ref[1] - w2_ref[0]], axis=0)
    sums2 = jnp.dot(qsum, _bd4(w2cat), precision=_HIGH)
    logits_p = sums2 * rc_ref[...] + r2_ref[...]
    m = jnp.max(logits_p, axis=1, keepdims=True)
    ex = jnp.exp(logits_p - m)
    r_i = lax.broadcasted_iota(jnp.int32, (128, 128), 0)
    c_i = lax.broadcasted_iota(jnp.int32, (128, 128), 1)
    msum = (r_i // 32 == c_i // 32).astype(jnp.float32)
    gsum = jnp.dot(ex, msum, precision=_HIGH)
    o_ref[...] = logits_p - m - jnp.log(gsum)


def kernel(x, edge_index, edge_attr, w1, root1, bias1, w2, root2, bias2):
    n, f_in = x.shape
    hid = w1.shape[2]
    ncls = w2.shape[2]
    npad4 = N_PAD // 4
    t1d = edge_attr.reshape(-1)
    x4 = jnp.pad(x.reshape(n // 4, 4 * f_in),
                 ((0, npad4 - n // 4), (0, 0)))

    grid = (npad4 // BRP,)
    u1p, xrp = pl.pallas_call(
        _tc_pre_body,
        grid=grid,
        in_specs=[
            pl.BlockSpec((BRP, 4 * f_in), lambda i: (i, 0)),
            pl.BlockSpec((2, f_in, hid), lambda i: (0, 0, 0)),
            pl.BlockSpec((f_in, hid), lambda i: (0, 0)),
        ],
        out_specs=(
            pl.BlockSpec((BRP, 128), lambda i: (i, 0)),
            pl.BlockSpec((BRP, 128), lambda i: (i, 0)),
        ),
        out_shape=(
            jax.ShapeDtypeStruct((npad4, 128), jnp.float32),
            jax.ShapeDtypeStruct((npad4, 128), jnp.float32),
        ),
    )(x4, w1, root1)

    p1 = _make_sc_scatter(1, SPLIT_L1)(u1p.reshape(N_PAD, 32),
                                       edge_index, t1d)

    hp, r2p, rcp = pl.pallas_call(
        _tc_mid_body,
        grid=grid,
        in_specs=[
            pl.BlockSpec((2, BRP, 128), lambda i: (0, i, 0)),
            pl.BlockSpec((BRP, 128), lambda i: (i, 0)),
            pl.BlockSpec((hid,), lambda i: (0,)),
            pl.BlockSpec((hid, ncls), lambda i: (0, 0)),
            pl.BlockSpec((ncls,), lambda i: (0,)),
        ],
        out_specs=(
            pl.BlockSpec((BRP, 128), lambda i: (i, 0)),
            pl.BlockSpec((BRP, 128), lambda i: (i, 0)),
            pl.BlockSpec((BRP, 128), lambda i: (i, 0)),
        ),
        out_shape=(
            jax.ShapeDtypeStruct((npad4, 128), jnp.float32),
            jax.ShapeDtypeStruct((npad4, 128), jnp.float32),
            jax.ShapeDtypeStruct((npad4, 128), jnp.float32),
        ),
    )(p1.reshape(NC, npad4, 128), xrp, bias1, root2, bias2)

    p2 = _make_sc_scatter(2, SPLIT_L2)(hp.reshape(N_PAD, 32),
                                       edge_index, t1d)

    outp = pl.pallas_call(
        _tc_post_body,
        grid=grid,
        in_specs=[
            pl.BlockSpec((2, BRP, 128), lambda i: (0, i, 0)),
            pl.BlockSpec((2, hid, ncls), lambda i: (0, 0, 0)),
            pl.BlockSpec((BRP, 128), lambda i: (i, 0)),
            pl.BlockSpec((BRP, 128), lambda i: (i, 0)),
        ],
        out_specs=pl.BlockSpec((BRP, 128), lambda i: (i, 0)),
        out_shape=jax.ShapeDtypeStruct((npad4, 128), jnp.float32),
    )(p2.reshape(NC, npad4, 128), w2, rcp, r2p)
    return outp.reshape(N_PAD, ncls)[:n]


# speedup vs baseline: 22.0178x; 1.0013x over previous
import functools

import jax
import jax.numpy as jnp
from jax import lax
from jax.experimental import pallas as pl
from jax.experimental.pallas import tpu as pltpu
from jax.experimental.pallas import tpu_sc as plsc

_HIGH = lax.Precision.HIGHEST

N_NODES = 10000
N_EDGES = 320000
NC = 2
NS = 16
CHUNK = 128
G_CHUNKS = N_EDGES // CHUNK
N_PAD = 10240
RPT = N_PAD // NS
ZROWS = 128
ACC_W = 32

SPLIT_L1 = (76, 80)
SPLIT_L2 = (76, 80)


def _sc_scatter_body(mode, a, b,
                     u_hbm, ei_hbm, ea_hbm, out_hbm,
                     src_v, dst_v, t_v, rows_v, msg_v, zb_v, acc_sh,
                     gsem0, gsem1, gsem2, gsem3, ssem0, ssem1):
    cid = lax.axis_index("c")
    sid = lax.axis_index("s")
    zvec = jnp.zeros((16,), jnp.float32)
    gsem = (gsem0, gsem1, gsem2, gsem3)
    ssem = (ssem0, ssem1)
    a4 = a + 4
    start0 = sid * a + 4 * jnp.minimum(sid, 1)
    start1 = NS * a + 4 + sid * b
    start = jnp.where(cid == 0, start0, start1)
    nquads = jnp.where(cid == 0, a // 4 + (sid < 1), b // 4)
    ncnt = 4 * nquads

    def _zero_zb(r, c):
        zb_v[r, pl.ds(0, 16)] = zvec
        zb_v[r, pl.ds(16, 16)] = zvec
        return c
    lax.fori_loop(0, ZROWS, _zero_zb, 0)
    for r in range(RPT // ZROWS):
        pltpu.sync_copy(zb_v, acc_sh.at[pl.ds(sid * RPT + r * ZROWS, ZROWS)])

    e0 = start * CHUNK

    @pl.when(cid == 0)
    def _stage_a():
        pltpu.sync_copy(ei_hbm.at[pl.ds(0, 1), pl.ds(e0, a4 * CHUNK)],
                        src_v.at[pl.ds(0, 1), pl.ds(0, a4 * CHUNK)])
        pltpu.sync_copy(ei_hbm.at[pl.ds(1, 1), pl.ds(e0, a4 * CHUNK)],
                        dst_v.at[pl.ds(0, 1), pl.ds(0, a4 * CHUNK)])
        pltpu.sync_copy(ea_hbm.at[pl.ds(start, a4)],
                        t_v.at[pl.ds(0, a4)])

    @pl.when(cid == 1)
    def _stage_b():
        pltpu.sync_copy(ei_hbm.at[pl.ds(0, 1), pl.ds(e0, b * CHUNK)],
                        src_v.at[pl.ds(0, 1), pl.ds(0, b * CHUNK)])
        pltpu.sync_copy(ei_hbm.at[pl.ds(1, 1), pl.ds(e0, b * CHUNK)],
                        dst_v.at[pl.ds(0, 1), pl.ds(0, b * CHUNK)])
        pltpu.sync_copy(ea_hbm.at[pl.ds(start, b)],
                        t_v.at[pl.ds(0, b)])

    if mode == 1:
        cnt_vec = (lax.iota(jnp.int32, 16) == 0).astype(jnp.float32)
        for s in range(2):
            for e in range(CHUNK):
                msg_v[s, e, pl.ds(16, 16)] = cnt_vec
    plsc.subcore_barrier()

    pltpu.async_copy(u_hbm.at[src_v.at[0, pl.ds(0, CHUNK)]], rows_v.at[0],
                     gsem[0])

    @pl.when(1 < ncnt)
    def _prime1():
        pltpu.async_copy(u_hbm.at[src_v.at[0, pl.ds(CHUNK, CHUNK)]],
                         rows_v.at[1], gsem[1])

    @pl.when(2 < ncnt)
    def _prime2():
        pltpu.async_copy(u_hbm.at[src_v.at[0, pl.ds(2 * CHUNK, CHUNK)]],
                         rows_v.at[2], gsem[2])

    def _quad(gg, c):
        for s in range(4):
            g = 4 * gg + s
            ms = s % 2
            nx2 = (s + 3) % 4

            pltpu.make_async_copy(
                u_hbm.at[src_v.at[0, pl.ds(g * CHUNK, CHUNK)]],
                rows_v.at[s], gsem[s]).wait()

            @pl.when(g + 3 < ncnt)
            def _issue():
                pltpu.async_copy(
                    u_hbm.at[src_v.at[0, pl.ds((g + 3) * CHUNK, CHUNK)]],
                    rows_v.at[nx2], gsem[nx2])

            @pl.when(g >= 2)
            def _drain_prev():
                pltpu.make_async_copy(
                    msg_v.at[ms],
                    acc_sh.at[dst_v.at[0, pl.ds(g * CHUNK, CHUNK)]],
                    ssem[ms]).wait()

            for jg in range(CHUNK // 16):
                tv = t_v[g, pl.ds(jg * 16, 16)]
                for j in range(16):
                    e = jg * 16 + j
                    if mode == 1:
                        z0 = rows_v[s, e, pl.ds(0, 16)]
                        z1 = rows_v[s, e, pl.ds(16, 16)]
                        msg_v[ms, e, pl.ds(0, 16)] = z0 + tv[j] * z1
                    else:
                        hrow = rows_v[s, e, pl.ds(0, 16)]
                        msg_v[ms, e, pl.ds(0, 16)] = hrow
                        msg_v[ms, e, pl.ds(16, 16)] = tv[j] * hrow
            pltpu.async_copy(msg_v.at[ms],
                             acc_sh.at[dst_v.at[0, pl.ds(g * CHUNK, CHUNK)]],
                             ssem[ms], add=True)
        return c
    lax.fori_loop(0, nquads, _quad, 0)

    for s in range(2):
        pltpu.make_async_copy(msg_v.at[s],
                              acc_sh.at[dst_v.at[0, pl.ds(0, CHUNK)]],
                              ssem[s]).wait()

    plsc.subcore_barrier()
    pltpu.sync_copy(acc_sh.at[pl.ds(sid * RPT, RPT)],
                    out_hbm.at[cid, pl.ds(sid * RPT, RPT)])


def _make_sc_scatter(mode, split):
    a, b = split
    ecap = (a + 4) * CHUNK
    row_w = 32
    mesh = plsc.VectorSubcoreMesh(core_axis_name="c", subcore_axis_name="s")
    return pl.kernel(
        functools.partial(_sc_scatter_body, mode, a, b),
        out_type=jax.ShapeDtypeStruct((NC, N_PAD, ACC_W), jnp.float32),
        mesh=mesh,
        scratch_types=[
            pltpu.VMEM((1, ecap), jnp.int32),
            pltpu.VMEM((1, ecap), jnp.int32),
            pltpu.VMEM((a + 4, CHUNK), jnp.float32),
            pltpu.VMEM((4, CHUNK, row_w), jnp.float32),
            pltpu.VMEM((2, CHUNK, ACC_W), jnp.float32),
            pltpu.VMEM((ZROWS, ACC_W), jnp.float32),
            pltpu.VMEM_SHARED((N_PAD, ACC_W), jnp.float32),
            pltpu.SemaphoreType.DMA,
            pltpu.SemaphoreType.DMA,
            pltpu.SemaphoreType.DMA,
            pltpu.SemaphoreType.DMA,
            pltpu.SemaphoreType.DMA,
            pltpu.SemaphoreType.DMA,
        ],
        compiler_params=pltpu.CompilerParams(
            use_tc_tiling_on_sc=False, needs_layout_passes=False,
            disable_bounds_checks=True),
    )


BRP = 320


def _bd4(m):
    z = jnp.zeros_like(m)
    rows = [jnp.concatenate([m if j == i else z for j in range(4)], axis=1)
            for i in range(4)]
    return jnp.concatenate(rows, axis=0)


def _tc_pre_body(x4_ref, w1_ref, root1_ref, u_ref, xr_ref):
    w_cat = jnp.concatenate([w1_ref[0], w1_ref[1] - w1_ref[0]], axis=1)
    r1pad = jnp.concatenate(
        [root1_ref[...], jnp.zeros_like(root1_ref[...])], axis=1)
    x4 = x4_ref[...]
    u_ref[...] = jnp.dot(x4, _bd4(w_cat), precision=_HIGH)
    xr_ref[...] = jnp.dot(x4, _bd4(r1pad), precision=_HIGH)


def _tc_mid_body(p_ref, xr_ref, bias1_ref, root2_ref, bias2_ref,
                 h_ref, r2_ref, rc_ref):
    psum = p_ref[0] + p_ref[1]
    r_i = lax.broadcasted_iota(jnp.int32, (128, 128), 0)
    c_i = lax.broadcasted_iota(jnp.int32, (128, 128), 1)
    mdiv = (r_i == (c_i // 32) * 32 + 16).astype(jnp.float32)
    divisor = jnp.maximum(jnp.dot(psum, mdiv, precision=_HIGH), 1.0)
    rc_ref[...] = 1.0 / divisor
    b1 = jnp.tile(jnp.concatenate([bias1_ref[...],
                                   jnp.zeros((16,), jnp.float32)]), 4)
    pre = psum / divisor + xr_ref[...] + b1
    h = jnp.where(pre > 0, pre, jnp.exp(jnp.minimum(pre, 0.0)) - 1.0)
    h_ref[...] = h
    r2pad = jnp.concatenate(
        [root2_ref[...], jnp.zeros_like(root2_ref[...])], axis=0)
    b2 = jnp.tile(bias2_ref[...], 4)
    r2_ref[...] = jnp.dot(h, _bd4(r2pad), precision=_HIGH) + b2


def _tc_post_body(q_ref, w2_ref, rc_ref, r2_ref, o_ref):
    qsum = q_ref[0] + q_ref[1]
    w2cat = jnp.concatenate([w2_ref[0], w2_ref[1] - w2_ref[0]], axis=0)
    sums2 = jnp.dot(qsum, _bd4(w2cat), precision=_HIGH)
    logits_p = sums2 * rc_ref[...] + r2_ref[...]
    m = jnp.max(logits_p, axis=1, keepdims=True)
    ex = jnp.exp(logits_p - m)
    r_i = lax.broadcasted_iota(jnp.int32, (128, 128), 0)
    c_i = lax.broadcasted_iota(jnp.int32, (128, 128), 1)
    msum = (r_i // 32 == c_i // 32).astype(jnp.float32)
    gsum = jnp.dot(ex, msum, precision=_HIGH)
    o_ref[...] = logits_p - m - jnp.log(gsum)


def kernel(x, edge_index, edge_attr, w1, root1, bias1, w2, root2, bias2):
    n, f_in = x.shape
    hid = w1.shape[2]
    ncls = w2.shape[2]
    npad4 = N_PAD // 4
    t2d = edge_attr.reshape(N_EDGES // CHUNK, CHUNK)
    x4 = jnp.pad(x.reshape(n // 4, 4 * f_in),
                 ((0, npad4 - n // 4), (0, 0)))

    grid = (npad4 // BRP,)
    u1p, xrp = pl.pallas_call(
        _tc_pre_body,
        grid=grid,
        in_specs=[
            pl.BlockSpec((BRP, 4 * f_in), lambda i: (i, 0)),
            pl.BlockSpec((2, f_in, hid), lambda i: (0, 0, 0)),
            pl.BlockSpec((f_in, hid), lambda i: (0, 0)),
        ],
        out_specs=(
            pl.BlockSpec((BRP, 128), lambda i: (i, 0)),
            pl.BlockSpec((BRP, 128), lambda i: (i, 0)),
        ),
        out_shape=(
            jax.ShapeDtypeStruct((npad4, 128), jnp.float32),
            jax.ShapeDtypeStruct((npad4, 128), jnp.float32),
        ),
    )(x4, w1, root1)

    p1 = _make_sc_scatter(1, SPLIT_L1)(u1p.reshape(N_PAD, 32),
                                       edge_index, t2d)

    hp, r2p, rcp = pl.pallas_call(
        _tc_mid_body,
        grid=grid,
        in_specs=[
            pl.BlockSpec((2, BRP, 128), lambda i: (0, i, 0)),
            pl.BlockSpec((BRP, 128), lambda i: (i, 0)),
            pl.BlockSpec((hid,), lambda i: (0,)),
            pl.BlockSpec((hid, ncls), lambda i: (0, 0)),
            pl.BlockSpec((ncls,), lambda i: (0,)),
        ],
        out_specs=(
            pl.BlockSpec((BRP, 128), lambda i: (i, 0)),
            pl.BlockSpec((BRP, 128), lambda i: (i, 0)),
            pl.BlockSpec((BRP, 128), lambda i: (i, 0)),
        ),
        out_shape=(
            jax.ShapeDtypeStruct((npad4, 128), jnp.float32),
            jax.ShapeDtypeStruct((npad4, 128), jnp.float32),
            jax.ShapeDtypeStruct((npad4, 128), jnp.float32),
        ),
    )(p1.reshape(NC, npad4, 128), xrp, bias1, root2, bias2)

    p2 = _make_sc_scatter(2, SPLIT_L2)(hp.reshape(N_PAD, 32),
                                       edge_index, t2d)

    outp = pl.pallas_call(
        _tc_post_body,
        grid=grid,
        in_specs=[
            pl.BlockSpec((2, BRP, 128), lambda i: (0, i, 0)),
            pl.BlockSpec((2, hid, ncls), lambda i: (0, 0, 0)),
            pl.BlockSpec((BRP, 128), lambda i: (i, 0)),
            pl.BlockSpec((BRP, 128), lambda i: (i, 0)),
        ],
        out_specs=pl.BlockSpec((BRP, 128), lambda i: (i, 0)),
        out_shape=jax.ShapeDtypeStruct((npad4, 128), jnp.float32),
    )(p2.reshape(NC, npad4, 128), w2, rcp, r2p)
    return outp.reshape(N_PAD, ncls)[:n]
